# baseline (speedup 1.0000x reference)
import functools

import jax
import jax.numpy as jnp
from jax import lax
from jax.experimental import pallas as pl
from jax.experimental.pallas import tpu as pltpu
from jax.experimental.pallas import tpu_sc as plsc

DM = 768
DF = 3072
NE = 8
NT = 2048
NV = 2
NA = 4
NTOT = NT * NA
BM = 256
MAXB = NTOT // BM + NE
MAXR = MAXB * BM
NB = NT // BM


def _routing_body(v_ref, pw_ref, pb_ref, rw_ref, keys_ref, h_ref, i_ref, g_ref):
    v = v_ref[0]
    h = jnp.dot(v, pw_ref[0], preferred_element_type=jnp.float32) + pb_ref[0]
    r = jnp.dot(h, rw_ref[0], preferred_element_type=jnp.float32)
    keys = keys_ref[...]
    diff = r[:, None, :] - keys[None, :, :]
    d2 = jnp.sum(diff * diff, axis=-1)
    logits = -jnp.sqrt(d2 + 1e-12)
    iota = lax.broadcasted_iota(jnp.int32, (BM, NE), 1)
    m1 = jnp.max(logits, axis=1, keepdims=True)
    i1 = jnp.min(jnp.where(logits == m1, iota, NE), axis=1, keepdims=True)
    l2 = jnp.where(iota == i1, -1e30, logits)
    m2 = jnp.max(l2, axis=1, keepdims=True)
    i2 = jnp.min(jnp.where(l2 == m2, iota, NE), axis=1, keepdims=True)
    e2 = jnp.exp(m2 - m1)
    den = 1.0 + e2
    h_ref[...] = h
    i_ref[...] = jnp.concatenate([i1, i2], axis=1)
    g_ref[...] = jnp.concatenate([1.0 / den, e2 / den], axis=1)


def _routing(V, proj_W, proj_b, router_W, expert_keys):
    return pl.pallas_call(
        _routing_body,
        grid=(NV, NB),
        in_specs=[
            pl.BlockSpec((1, BM, DM), lambda v, t: (v, t, 0)),
            pl.BlockSpec((1, DM, DM), lambda v, t: (v, 0, 0)),
            pl.BlockSpec((1, 1, DM), lambda v, t: (v, 0, 0)),
            pl.BlockSpec((1, DM, NE), lambda v, t: (v, 0, 0)),
            pl.BlockSpec((NE, NE), lambda v, t: (0, 0)),
        ],
        out_specs=[
            pl.BlockSpec((BM, DM), lambda v, t: (v * NB + t, 0)),
            pl.BlockSpec((BM, 2), lambda v, t: (v * NB + t, 0)),
            pl.BlockSpec((BM, 2), lambda v, t: (v * NB + t, 0)),
        ],
        out_shape=[
            jax.ShapeDtypeStruct((NV * NT, DM), jnp.float32),
            jax.ShapeDtypeStruct((NV * NT, 2), jnp.int32),
            jax.ShapeDtypeStruct((NV * NT, 2), jnp.float32),
        ],
    )(V, proj_W, proj_b.reshape(NV, 1, DM), router_W, expert_keys)


def _dispatch_body(idx_ref, gv_ref, dest_ref, gk_ref, be_ref):
    ek = jnp.concatenate([idx_ref[:NT, :], idx_ref[NT:, :]], axis=1)
    gk = jnp.concatenate([gv_ref[:NT, :], gv_ref[NT:, :]], axis=1)
    iota_e = lax.broadcasted_iota(jnp.int32, (NT, NE), 1)

    cnt_row = jnp.zeros((NT, NE), jnp.float32)
    for c in range(NA):
        cnt_row = cnt_row + (ek[:, c:c + 1] == iota_e).astype(jnp.float32)

    ltri = (lax.broadcasted_iota(jnp.int32, (NT, NT), 0)
            > lax.broadcasted_iota(jnp.int32, (NT, NT), 1)).astype(jnp.float32)
    cbefore = jnp.dot(ltri, cnt_row, preferred_element_type=jnp.float32)

    cnt = jnp.sum(cnt_row, axis=0, keepdims=True)
    cp = jnp.floor((cnt + (BM - 1)) * (1.0 / BM)) * BM
    x = cp
    for s in (1, 2, 4):
        x = x + jnp.concatenate(
            [jnp.zeros((1, s), jnp.float32), x[:, :NE - s]], axis=1)
    pad_off = x - cp

    dest_cols = []
    for c in range(NA):
        sel = (ek[:, c:c + 1] == iota_e).astype(jnp.float32)
        base = jnp.sum(sel * (cbefore + pad_off), axis=1, keepdims=True)
        wr = jnp.zeros((NT, 1), jnp.float32)
        for cprev in range(c):
            wr = wr + (ek[:, cprev:cprev + 1] == ek[:, c:c + 1]).astype(jnp.float32)
        dest_cols.append(base + wr)
    dest_ref[...] = jnp.concatenate(dest_cols, axis=1).astype(jnp.int32)
    gk_ref[...] = gk

    ends = jnp.broadcast_to(pad_off + cp, (MAXB, NE))
    bio = lax.broadcasted_iota(jnp.int32, (MAXB, NE), 0).astype(jnp.float32) * BM
    be = jnp.sum((ends <= bio).astype(jnp.int32), axis=1, keepdims=True)
    be_ref[...] = jnp.minimum(be, NE - 1)


def _dispatch(IDX, GV):
    return pl.pallas_call(
        _dispatch_body,
        out_shape=[
            jax.ShapeDtypeStruct((NT, NA), jnp.int32),
            jax.ShapeDtypeStruct((NT, NA), jnp.float32),
            jax.ShapeDtypeStruct((MAXB, 1), jnp.int32),
        ],
    )(IDX, GV)


def _scatter_kernel(dest_hbm, gk_hbm, srcrow_hbm, gates_hbm,
                    dest_v, gk_v, sr_v, gt_v):
    wid = lax.axis_index("s") * 2 + lax.axis_index("c")

    @pl.when(wid == 0)
    def _():
        pltpu.sync_copy(dest_hbm, dest_v)
        pltpu.sync_copy(gk_hbm, gk_v)

        def zero(i, _):
            sr_v[pl.ds(i * 16, 16)] = jnp.zeros((16,), jnp.int32)
            gt_v[pl.ds(i * 16, 16)] = jnp.zeros((16,), jnp.float32)
            return 0
        lax.fori_loop(0, MAXR // 16, zero, 0)

        def body(i, _):
            j0 = i * 16
            d = dest_v[pl.ds(j0, 16)]
            g = gt = gk_v[pl.ds(j0, 16)]
            j = lax.iota(jnp.int32, 16) + j0
            src = (j >> 2) + jnp.where((j & 3) >= 2, NT, 0)
            plsc.store_scatter(sr_v, [d], src)
            plsc.store_scatter(gt_v, [d], gt)
            return 0
        lax.fori_loop(0, NTOT // 16, body, 0)

        pltpu.sync_copy(sr_v, srcrow_hbm)
        pltpu.sync_copy(gt_v, gates_hbm)


def _scatter(dest_flat, gk_flat):
    mesh = plsc.VectorSubcoreMesh(core_axis_name="c", subcore_axis_name="s")
    return pl.kernel(
        _scatter_kernel,
        mesh=mesh,
        compiler_params=pltpu.CompilerParams(needs_layout_passes=False),
        out_type=[
            jax.ShapeDtypeStruct((MAXR,), jnp.int32),
            jax.ShapeDtypeStruct((MAXR,), jnp.float32),
        ],
        scratch_types=[
            pltpu.VMEM((NTOT,), jnp.int32),
            pltpu.VMEM((NTOT,), jnp.float32),
            pltpu.VMEM((MAXR,), jnp.int32),
            pltpu.VMEM((MAXR,), jnp.float32),
        ],
    )(dest_flat, gk_flat)


def _make_gather(n_rows, n_src_rows):
    per_w = n_rows // 32
    ch = 64
    n_ch = per_w // ch
    mesh = plsc.VectorSubcoreMesh(core_axis_name="c", subcore_axis_name="s")

    def body(idx_hbm, src_hbm, out_hbm, idx_v, rows_v, sem):
        wid = lax.axis_index("s") * 2 + lax.axis_index("c")
        base = wid * per_w
        for q in range(n_ch):
            o = base + q * ch
            pltpu.sync_copy(idx_hbm.at[pl.ds(o, ch)], idx_v)
            pltpu.async_copy(src_hbm.at[idx_v], rows_v, sem).wait()
            pltpu.sync_copy(rows_v, out_hbm.at[pl.ds(o, ch)])

    return pl.kernel(
        body,
        mesh=mesh,
        out_type=jax.ShapeDtypeStruct((n_rows, DM), jnp.float32),
        scratch_types=[
            pltpu.VMEM((ch,), jnp.int32),
            pltpu.VMEM((ch, DM), jnp.float32),
            pltpu.SemaphoreType.DMA,
        ],
    )


def _ffn_body(be_ref, xs_ref, w1_ref, b1_ref, w2_ref, b2_ref, g_ref, out_ref):
    x = xs_ref[...]
    mid = jnp.dot(x, w1_ref[0], preferred_element_type=jnp.float32) + b1_ref[0]
    mid = 0.5 * mid * (1.0 + lax.erf(mid * 0.7071067811865476))
    y = jnp.dot(mid, w2_ref[0], preferred_element_type=jnp.float32) + b2_ref[0]
    out_ref[...] = g_ref[...] * y


def _ffn(be, Xs, W1, b1, W2, b2, gates):
    grid_spec = pltpu.PrefetchScalarGridSpec(
        num_scalar_prefetch=1,
        grid=(MAXB,),
        in_specs=[
            pl.BlockSpec((BM, DM), lambda b, be_ref: (b, 0)),
            pl.BlockSpec((1, DM, DF), lambda b, be_ref: (be_ref[b], 0, 0)),
            pl.BlockSpec((1, 1, DF), lambda b, be_ref: (be_ref[b], 0, 0)),
            pl.BlockSpec((1, DF, DM), lambda b, be_ref: (be_ref[b], 0, 0)),
            pl.BlockSpec((1, 1, DM), lambda b, be_ref: (be_ref[b], 0, 0)),
            pl.BlockSpec((BM, 1), lambda b, be_ref: (b, 0)),
        ],
        out_specs=pl.BlockSpec((BM, DM), lambda b, be_ref: (b, 0)),
    )
    return pl.pallas_call(
        _ffn_body,
        grid_spec=grid_spec,
        out_shape=jax.ShapeDtypeStruct((MAXR, DM), jnp.float32),
    )(be, Xs, W1, b1.reshape(NE, 1, DF), W2, b2.reshape(NE, 1, DM),
      gates.reshape(MAXR, 1))


def _combine_body(z_ref, out_ref):
    out_ref[...] = jnp.sum(z_ref[...], axis=1)


def _combine(Z):
    return pl.pallas_call(
        _combine_body,
        grid=(NB,),
        in_specs=[pl.BlockSpec((BM, NA, DM), lambda t: (t, 0, 0))],
        out_specs=pl.BlockSpec((BM, DM), lambda t: (t, 0)),
        out_shape=jax.ShapeDtypeStruct((NT, DM), jnp.float32),
    )(Z)


def kernel(view0, view1, proj_W, proj_b, router_W, expert_keys, W1, b1, W2, b2):
    V = jnp.concatenate([view0, view1], axis=0)
    H, IDX, GV = _routing(V, proj_W, proj_b, router_W, expert_keys)
    dest, gk, be = _dispatch(IDX, GV)
    dest_flat = dest.reshape(NTOT)
    srcrow, gates = _scatter(dest_flat, gk.reshape(NTOT))
    Xs = _make_gather(MAXR, NV * NT)(srcrow, H)
    Ys = _ffn(be.reshape(MAXB), Xs, W1, b1, W2, b2, gates)
    Z = _make_gather(NTOT, MAXR)(dest_flat, Ys)
    out = _combine(Z.reshape(NT, NA, DM))
    return out.reshape(1, NT, DM)


# speedup vs baseline: 1.0061x; 1.0061x over previous
import functools

import jax
import jax.numpy as jnp
from jax import lax
from jax.experimental import pallas as pl
from jax.experimental.pallas import tpu as pltpu
from jax.experimental.pallas import tpu_sc as plsc

DM = 768
DF = 3072
NE = 8
NT = 2048
NV = 2
NA = 4
NTOT = NT * NA
BM = 256
MAXB = NTOT // BM + NE
MAXR = MAXB * BM
NB = NT // BM


def _routing_body(v_ref, pw_ref, pb_ref, rw_ref, keys_ref, h_ref, i_ref, g_ref):
    v = v_ref[0]
    h = jnp.dot(v, pw_ref[0], preferred_element_type=jnp.float32) + pb_ref[0]
    r = jnp.dot(h, rw_ref[0], preferred_element_type=jnp.float32)
    keys = keys_ref[...]
    diff = r[:, None, :] - keys[None, :, :]
    d2 = jnp.sum(diff * diff, axis=-1)
    logits = -jnp.sqrt(d2 + 1e-12)
    iota = lax.broadcasted_iota(jnp.int32, (BM, NE), 1)
    m1 = jnp.max(logits, axis=1, keepdims=True)
    i1 = jnp.min(jnp.where(logits == m1, iota, NE), axis=1, keepdims=True)
    l2 = jnp.where(iota == i1, -1e30, logits)
    m2 = jnp.max(l2, axis=1, keepdims=True)
    i2 = jnp.min(jnp.where(l2 == m2, iota, NE), axis=1, keepdims=True)
    e2 = jnp.exp(m2 - m1)
    den = 1.0 + e2
    h_ref[...] = h
    i_ref[...] = jnp.concatenate([i1, i2], axis=1)
    g_ref[...] = jnp.concatenate([1.0 / den, e2 / den], axis=1)


def _routing(V, proj_W, proj_b, router_W, expert_keys):
    return pl.pallas_call(
        _routing_body,
        grid=(NV, NB),
        in_specs=[
            pl.BlockSpec((1, BM, DM), lambda v, t: (v, t, 0)),
            pl.BlockSpec((1, DM, DM), lambda v, t: (v, 0, 0)),
            pl.BlockSpec((1, 1, DM), lambda v, t: (v, 0, 0)),
            pl.BlockSpec((1, DM, NE), lambda v, t: (v, 0, 0)),
            pl.BlockSpec((NE, NE), lambda v, t: (0, 0)),
        ],
        out_specs=[
            pl.BlockSpec((BM, DM), lambda v, t: (v * NB + t, 0)),
            pl.BlockSpec((BM, 2), lambda v, t: (v * NB + t, 0)),
            pl.BlockSpec((BM, 2), lambda v, t: (v * NB + t, 0)),
        ],
        out_shape=[
            jax.ShapeDtypeStruct((NV * NT, DM), jnp.float32),
            jax.ShapeDtypeStruct((NV * NT, 2), jnp.int32),
            jax.ShapeDtypeStruct((NV * NT, 2), jnp.float32),
        ],
    )(V, proj_W, proj_b.reshape(NV, 1, DM), router_W, expert_keys)


def _dispatch_body(idx_ref, gv_ref, dest_ref, gk_ref, be_ref):
    ek = jnp.concatenate([idx_ref[:NT, :], idx_ref[NT:, :]], axis=1)
    gk = jnp.concatenate([gv_ref[:NT, :], gv_ref[NT:, :]], axis=1)
    iota_e = lax.broadcasted_iota(jnp.int32, (NT, NE), 1)

    cnt_row = jnp.zeros((NT, NE), jnp.float32)
    for c in range(NA):
        cnt_row = cnt_row + (ek[:, c:c + 1] == iota_e).astype(jnp.float32)

    ltri = (lax.broadcasted_iota(jnp.int32, (NT, NT), 0)
            > lax.broadcasted_iota(jnp.int32, (NT, NT), 1)).astype(jnp.float32)
    cbefore = jnp.dot(ltri, cnt_row, preferred_element_type=jnp.float32)

    cnt = jnp.sum(cnt_row, axis=0, keepdims=True)
    cp = jnp.floor((cnt + (BM - 1)) * (1.0 / BM)) * BM
    x = cp
    for s in (1, 2, 4):
        x = x + jnp.concatenate(
            [jnp.zeros((1, s), jnp.float32), x[:, :NE - s]], axis=1)
    pad_off = x - cp

    dest_cols = []
    for c in range(NA):
        sel = (ek[:, c:c + 1] == iota_e).astype(jnp.float32)
        base = jnp.sum(sel * (cbefore + pad_off), axis=1, keepdims=True)
        wr = jnp.zeros((NT, 1), jnp.float32)
        for cprev in range(c):
            wr = wr + (ek[:, cprev:cprev + 1] == ek[:, c:c + 1]).astype(jnp.float32)
        dest_cols.append(base + wr)
    dest_ref[...] = jnp.concatenate(dest_cols, axis=1).astype(jnp.int32)
    gk_ref[...] = gk

    ends = jnp.broadcast_to(pad_off + cp, (MAXB, NE))
    bio = lax.broadcasted_iota(jnp.int32, (MAXB, NE), 0).astype(jnp.float32) * BM
    be = jnp.sum((ends <= bio).astype(jnp.int32), axis=1, keepdims=True)
    be_ref[...] = jnp.minimum(be, NE - 1)


def _dispatch(IDX, GV):
    return pl.pallas_call(
        _dispatch_body,
        out_shape=[
            jax.ShapeDtypeStruct((NT, NA), jnp.int32),
            jax.ShapeDtypeStruct((NT, NA), jnp.float32),
            jax.ShapeDtypeStruct((MAXB, 1), jnp.int32),
        ],
    )(IDX, GV)


def _scatter_kernel(dest_hbm, gk_hbm, zi_hbm, zf_hbm, srcrow_hbm, gates_hbm,
                    dest_v, gk_v, sr_v, gt_v):
    wid = lax.axis_index("s") * 2 + lax.axis_index("c")

    @pl.when(wid == 0)
    def _():
        pltpu.sync_copy(dest_hbm, dest_v)
        pltpu.sync_copy(gk_hbm, gk_v)
        pltpu.sync_copy(zi_hbm, sr_v)
        pltpu.sync_copy(zf_hbm, gt_v)

        @plsc.parallel_loop(0, NTOT // 16, unroll=4)
        def _body(i):
            j0 = i * 16
            d = dest_v[pl.ds(j0, 16)]
            gt = gk_v[pl.ds(j0, 16)]
            j = lax.iota(jnp.int32, 16) + j0
            src = (j >> 2) + jnp.where((j & 3) >= 2, NT, 0)
            plsc.store_scatter(sr_v, [d], src)
            plsc.store_scatter(gt_v, [d], gt)

        pltpu.sync_copy(sr_v, srcrow_hbm)
        pltpu.sync_copy(gt_v, gates_hbm)


def _scatter(dest_flat, gk_flat):
    mesh = plsc.VectorSubcoreMesh(core_axis_name="c", subcore_axis_name="s")
    return pl.kernel(
        _scatter_kernel,
        mesh=mesh,
        compiler_params=pltpu.CompilerParams(needs_layout_passes=False),
        out_type=[
            jax.ShapeDtypeStruct((MAXR,), jnp.int32),
            jax.ShapeDtypeStruct((MAXR,), jnp.float32),
        ],
        scratch_types=[
            pltpu.VMEM((NTOT,), jnp.int32),
            pltpu.VMEM((NTOT,), jnp.float32),
            pltpu.VMEM((MAXR,), jnp.int32),
            pltpu.VMEM((MAXR,), jnp.float32),
        ],
    )(dest_flat, gk_flat,
      jnp.zeros((MAXR,), jnp.int32), jnp.zeros((MAXR,), jnp.float32))


def _make_gather(n_rows, n_src_rows):
    per_w = n_rows // 32
    ch = 64
    n_ch = per_w // ch
    mesh = plsc.VectorSubcoreMesh(core_axis_name="c", subcore_axis_name="s")

    def body(idx_hbm, src_hbm, out_hbm, idx_v, buf0, buf1, sg0, sg1, sw0, sw1):
        wid = lax.axis_index("s") * 2 + lax.axis_index("c")
        base = wid * per_w
        pltpu.sync_copy(idx_hbm.at[pl.ds(base, per_w)], idx_v)
        bufs = (buf0, buf1)
        sgs = (sg0, sg1)
        sws = (sw0, sw1)

        def start_gather(q):
            return pltpu.async_copy(
                src_hbm.at[idx_v.at[pl.ds(q * ch, ch)]], bufs[q % 2], sgs[q % 2])

        def start_write(q):
            return pltpu.async_copy(
                bufs[q % 2], out_hbm.at[pl.ds(base + q * ch, ch)], sws[q % 2])

        dg = {0: start_gather(0)}
        dw = {}
        for q in range(n_ch):
            if q + 1 < n_ch:
                if q >= 1:
                    dw[q - 1].wait()
                dg[q + 1] = start_gather(q + 1)
            dg[q].wait()
            dw[q] = start_write(q)
        dw[n_ch - 1].wait()
        if n_ch >= 2:
            dw[n_ch - 2].wait()

    return pl.kernel(
        body,
        mesh=mesh,
        out_type=jax.ShapeDtypeStruct((n_rows, DM), jnp.float32),
        scratch_types=[
            pltpu.VMEM((per_w,), jnp.int32),
            pltpu.VMEM((ch, DM), jnp.float32),
            pltpu.VMEM((ch, DM), jnp.float32),
            pltpu.SemaphoreType.DMA,
            pltpu.SemaphoreType.DMA,
            pltpu.SemaphoreType.DMA,
            pltpu.SemaphoreType.DMA,
        ],
    )


def _ffn_body(be_ref, xs_ref, w1_ref, b1_ref, w2_ref, b2_ref, g_ref, out_ref):
    x = xs_ref[...]
    mid = jnp.dot(x, w1_ref[0], preferred_element_type=jnp.float32) + b1_ref[0]
    mid = 0.5 * mid * (1.0 + lax.erf(mid * 0.7071067811865476))
    y = jnp.dot(mid, w2_ref[0], preferred_element_type=jnp.float32) + b2_ref[0]
    out_ref[...] = g_ref[...] * y


def _ffn(be, Xs, W1, b1, W2, b2, gates):
    grid_spec = pltpu.PrefetchScalarGridSpec(
        num_scalar_prefetch=1,
        grid=(MAXB,),
        in_specs=[
            pl.BlockSpec((BM, DM), lambda b, be_ref: (b, 0)),
            pl.BlockSpec((1, DM, DF), lambda b, be_ref: (be_ref[b], 0, 0)),
            pl.BlockSpec((1, 1, DF), lambda b, be_ref: (be_ref[b], 0, 0)),
            pl.BlockSpec((1, DF, DM), lambda b, be_ref: (be_ref[b], 0, 0)),
            pl.BlockSpec((1, 1, DM), lambda b, be_ref: (be_ref[b], 0, 0)),
            pl.BlockSpec((BM, 1), lambda b, be_ref: (b, 0)),
        ],
        out_specs=pl.BlockSpec((BM, DM), lambda b, be_ref: (b, 0)),
    )
    return pl.pallas_call(
        _ffn_body,
        grid_spec=grid_spec,
        out_shape=jax.ShapeDtypeStruct((MAXR, DM), jnp.float32),
    )(be, Xs, W1, b1.reshape(NE, 1, DF), W2, b2.reshape(NE, 1, DM),
      gates.reshape(MAXR, 1))


def _combine_body(z_ref, out_ref):
    out_ref[...] = jnp.sum(z_ref[...], axis=1)


def _combine(Z):
    return pl.pallas_call(
        _combine_body,
        grid=(NB,),
        in_specs=[pl.BlockSpec((BM, NA, DM), lambda t: (t, 0, 0))],
        out_specs=pl.BlockSpec((BM, DM), lambda t: (t, 0)),
        out_shape=jax.ShapeDtypeStruct((NT, DM), jnp.float32),
    )(Z)


def kernel(view0, view1, proj_W, proj_b, router_W, expert_keys, W1, b1, W2, b2):
    V = jnp.concatenate([view0, view1], axis=0)
    H, IDX, GV = _routing(V, proj_W, proj_b, router_W, expert_keys)
    dest, gk, be = _dispatch(IDX, GV)
    dest_flat = dest.reshape(NTOT)
    srcrow, gates = _scatter(dest_flat, gk.reshape(NTOT))
    Xs = _make_gather(MAXR, NV * NT)(srcrow, H)
    Ys = _ffn(be.reshape(MAXB), Xs, W1, b1, W2, b2, gates)
    Z = _make_gather(NTOT, MAXR)(dest_flat, Ys)
    out = _combine(Z.reshape(NT, NA, DM))
    return out.reshape(1, NT, DM)


# speedup vs baseline: 1.2921x; 1.2843x over previous
import functools

import jax
import jax.numpy as jnp
from jax import lax
from jax.experimental import pallas as pl
from jax.experimental.pallas import tpu as pltpu
from jax.experimental.pallas import tpu_sc as plsc

DM = 768
DF = 3072
NE = 8
NT = 2048
NV = 2
NA = 4
NTOT = NT * NA
BM = 256
MAXB = NTOT // BM + NE
MAXR = MAXB * BM
NB = NT // BM


def _routing_body(v_ref, pw_ref, pb_ref, rw_ref, keys_ref, h_ref, i_ref, g_ref):
    v = v_ref[0]
    h = jnp.dot(v, pw_ref[0], preferred_element_type=jnp.float32) + pb_ref[0]
    r = jnp.dot(h, rw_ref[0], preferred_element_type=jnp.float32)
    keys = keys_ref[...]
    diff = r[:, None, :] - keys[None, :, :]
    d2 = jnp.sum(diff * diff, axis=-1)
    logits = -jnp.sqrt(d2 + 1e-12)
    iota = lax.broadcasted_iota(jnp.int32, (BM, NE), 1)
    m1 = jnp.max(logits, axis=1, keepdims=True)
    i1 = jnp.min(jnp.where(logits == m1, iota, NE), axis=1, keepdims=True)
    l2 = jnp.where(iota == i1, -1e30, logits)
    m2 = jnp.max(l2, axis=1, keepdims=True)
    i2 = jnp.min(jnp.where(l2 == m2, iota, NE), axis=1, keepdims=True)
    e2 = jnp.exp(m2 - m1)
    den = 1.0 + e2
    h_ref[...] = h
    i_ref[...] = jnp.concatenate([i1, i2], axis=1)
    g_ref[...] = jnp.concatenate([1.0 / den, e2 / den], axis=1)


def _routing(V, proj_W, proj_b, router_W, expert_keys):
    return pl.pallas_call(
        _routing_body,
        grid=(NV, NB),
        in_specs=[
            pl.BlockSpec((1, BM, DM), lambda v, t: (v, t, 0)),
            pl.BlockSpec((1, DM, DM), lambda v, t: (v, 0, 0)),
            pl.BlockSpec((1, 1, DM), lambda v, t: (v, 0, 0)),
            pl.BlockSpec((1, DM, NE), lambda v, t: (v, 0, 0)),
            pl.BlockSpec((NE, NE), lambda v, t: (0, 0)),
        ],
        out_specs=[
            pl.BlockSpec((BM, DM), lambda v, t: (v * NB + t, 0)),
            pl.BlockSpec((BM, 2), lambda v, t: (v * NB + t, 0)),
            pl.BlockSpec((BM, 2), lambda v, t: (v * NB + t, 0)),
        ],
        out_shape=[
            jax.ShapeDtypeStruct((NV * NT, DM), jnp.float32),
            jax.ShapeDtypeStruct((NV * NT, 2), jnp.int32),
            jax.ShapeDtypeStruct((NV * NT, 2), jnp.float32),
        ],
    )(V, proj_W, proj_b.reshape(NV, 1, DM), router_W, expert_keys)


def _dispatch_body(idx_ref, gv_ref, dest_ref, gk_ref, be_ref):
    ek = jnp.concatenate([idx_ref[:NT, :], idx_ref[NT:, :]], axis=1)
    gk = jnp.concatenate([gv_ref[:NT, :], gv_ref[NT:, :]], axis=1)
    iota_e = lax.broadcasted_iota(jnp.int32, (NT, NE), 1)

    cnt_row = jnp.zeros((NT, NE), jnp.float32)
    for c in range(NA):
        cnt_row = cnt_row + (ek[:, c:c + 1] == iota_e).astype(jnp.float32)

    ltri = (lax.broadcasted_iota(jnp.int32, (NT, NT), 0)
            > lax.broadcasted_iota(jnp.int32, (NT, NT), 1)).astype(jnp.float32)
    cbefore = jnp.dot(ltri, cnt_row, preferred_element_type=jnp.float32)

    cnt = jnp.sum(cnt_row, axis=0, keepdims=True)
    cp = jnp.floor((cnt + (BM - 1)) * (1.0 / BM)) * BM
    x = cp
    for s in (1, 2, 4):
        x = x + jnp.concatenate(
            [jnp.zeros((1, s), jnp.float32), x[:, :NE - s]], axis=1)
    pad_off = x - cp

    dest_cols = []
    for c in range(NA):
        sel = (ek[:, c:c + 1] == iota_e).astype(jnp.float32)
        base = jnp.sum(sel * (cbefore + pad_off), axis=1, keepdims=True)
        wr = jnp.zeros((NT, 1), jnp.float32)
        for cprev in range(c):
            wr = wr + (ek[:, cprev:cprev + 1] == ek[:, c:c + 1]).astype(jnp.float32)
        dest_cols.append(base + wr)
    dest_ref[...] = jnp.concatenate(dest_cols, axis=1).astype(jnp.int32)
    gk_ref[...] = gk

    ends = jnp.broadcast_to(pad_off + cp, (MAXB, NE))
    bio = lax.broadcasted_iota(jnp.int32, (MAXB, NE), 0).astype(jnp.float32) * BM
    be = jnp.sum((ends <= bio).astype(jnp.int32), axis=1, keepdims=True)
    nb_used = jnp.sum(cp * (1.0 / BM), axis=1, keepdims=True).astype(jnp.int32)
    be_ref[...] = jnp.concatenate([jnp.minimum(be, NE - 1), nb_used], axis=0)


def _dispatch(IDX, GV):
    return pl.pallas_call(
        _dispatch_body,
        out_shape=[
            jax.ShapeDtypeStruct((NT, NA), jnp.int32),
            jax.ShapeDtypeStruct((NT, NA), jnp.float32),
            jax.ShapeDtypeStruct((MAXB + 1, 1), jnp.int32),
        ],
    )(IDX, GV)


def _scatter_kernel(dest_hbm, gk_hbm, zi_hbm, zf_hbm, srcrow_hbm, gates_hbm,
                    dest_v, gk_v, sr_v, gt_v):
    wid = lax.axis_index("s") * 2 + lax.axis_index("c")

    @pl.when(wid == 0)
    def _():
        pltpu.sync_copy(dest_hbm, dest_v)
        pltpu.sync_copy(gk_hbm, gk_v)
        pltpu.sync_copy(zi_hbm, sr_v)
        pltpu.sync_copy(zf_hbm, gt_v)

        @plsc.parallel_loop(0, NTOT // 16, unroll=4)
        def _body(i):
            j0 = i * 16
            d = dest_v[pl.ds(j0, 16)]
            gt = gk_v[pl.ds(j0, 16)]
            j = lax.iota(jnp.int32, 16) + j0
            src = (j >> 2) + jnp.where((j & 3) >= 2, NT, 0)
            plsc.store_scatter(sr_v, [d], src)
            plsc.store_scatter(gt_v, [d], gt)

        pltpu.sync_copy(sr_v, srcrow_hbm)
        pltpu.sync_copy(gt_v, gates_hbm)


def _scatter(dest_flat, gk_flat):
    mesh = plsc.VectorSubcoreMesh(core_axis_name="c", subcore_axis_name="s")
    return pl.kernel(
        _scatter_kernel,
        mesh=mesh,
        compiler_params=pltpu.CompilerParams(needs_layout_passes=False),
        out_type=[
            jax.ShapeDtypeStruct((MAXR,), jnp.int32),
            jax.ShapeDtypeStruct((MAXR,), jnp.float32),
        ],
        scratch_types=[
            pltpu.VMEM((NTOT,), jnp.int32),
            pltpu.VMEM((NTOT,), jnp.float32),
            pltpu.VMEM((MAXR,), jnp.int32),
            pltpu.VMEM((MAXR,), jnp.float32),
        ],
    )(dest_flat, gk_flat,
      jnp.arange(MAXR, dtype=jnp.int32) % (NV * NT),
      jnp.zeros((MAXR,), jnp.float32))


def _make_gather(n_rows, n_src_rows):
    per_w = n_rows // 32
    ch = 64
    n_ch = per_w // ch
    mesh = plsc.VectorSubcoreMesh(core_axis_name="c", subcore_axis_name="s")

    def body(idx_hbm, src_hbm, out_hbm, idx_v, buf0, buf1, sg0, sg1, sw0, sw1):
        wid = lax.axis_index("s") * 2 + lax.axis_index("c")
        base = wid * per_w
        pltpu.sync_copy(idx_hbm.at[pl.ds(base, per_w)], idx_v)
        bufs = (buf0, buf1)
        sgs = (sg0, sg1)
        sws = (sw0, sw1)

        def start_gather(q):
            return pltpu.async_copy(
                src_hbm.at[idx_v.at[pl.ds(q * ch, ch)]], bufs[q % 2], sgs[q % 2])

        def start_write(q):
            return pltpu.async_copy(
                bufs[q % 2], out_hbm.at[pl.ds(base + q * ch, ch)], sws[q % 2])

        dg = {0: start_gather(0)}
        dw = {}
        for q in range(n_ch):
            if q + 1 < n_ch:
                if q >= 1:
                    dw[q - 1].wait()
                dg[q + 1] = start_gather(q + 1)
            dg[q].wait()
            dw[q] = start_write(q)
        dw[n_ch - 1].wait()
        if n_ch >= 2:
            dw[n_ch - 2].wait()

    return pl.kernel(
        body,
        mesh=mesh,
        out_type=jax.ShapeDtypeStruct((n_rows, DM), jnp.float32),
        scratch_types=[
            pltpu.VMEM((per_w,), jnp.int32),
            pltpu.VMEM((ch, DM), jnp.float32),
            pltpu.VMEM((ch, DM), jnp.float32),
            pltpu.SemaphoreType.DMA,
            pltpu.SemaphoreType.DMA,
            pltpu.SemaphoreType.DMA,
            pltpu.SemaphoreType.DMA,
        ],
    )


def _ffn_body(be_ref, xs_ref, w1_ref, b1_ref, w2_ref, b2_ref, g_ref, out_ref):
    b = pl.program_id(0)

    @pl.when(b < be_ref[MAXB])
    def _():
        x = xs_ref[...].astype(jnp.bfloat16)
        mid = jnp.dot(x, w1_ref[0], preferred_element_type=jnp.float32) + b1_ref[0]
        mid = 0.5 * mid * (1.0 + lax.erf(mid * 0.7071067811865476))
        y = jnp.dot(mid.astype(jnp.bfloat16), w2_ref[0],
                    preferred_element_type=jnp.float32) + b2_ref[0]
        out_ref[...] = g_ref[...] * y

    @pl.when(b >= be_ref[MAXB])
    def _():
        out_ref[...] = jnp.zeros_like(out_ref)


def _ffn(be, Xs, W1, b1, W2, b2, gates):
    grid_spec = pltpu.PrefetchScalarGridSpec(
        num_scalar_prefetch=1,
        grid=(MAXB,),
        in_specs=[
            pl.BlockSpec((BM, DM), lambda b, be_ref: (b, 0)),
            pl.BlockSpec((1, DM, DF), lambda b, be_ref: (be_ref[b], 0, 0)),
            pl.BlockSpec((1, 1, DF), lambda b, be_ref: (be_ref[b], 0, 0)),
            pl.BlockSpec((1, DF, DM), lambda b, be_ref: (be_ref[b], 0, 0)),
            pl.BlockSpec((1, 1, DM), lambda b, be_ref: (be_ref[b], 0, 0)),
            pl.BlockSpec((BM, 1), lambda b, be_ref: (b, 0)),
        ],
        out_specs=pl.BlockSpec((BM, DM), lambda b, be_ref: (b, 0)),
    )
    return pl.pallas_call(
        _ffn_body,
        grid_spec=grid_spec,
        out_shape=jax.ShapeDtypeStruct((MAXR, DM), jnp.float32),
    )(be, Xs, W1.astype(jnp.bfloat16), b1.reshape(NE, 1, DF),
      W2.astype(jnp.bfloat16), b2.reshape(NE, 1, DM),
      gates.reshape(MAXR, 1))


def _combine_body(z_ref, out_ref):
    x = z_ref[...].reshape(BM, NA, DM)
    out_ref[...] = jnp.sum(x, axis=1)


def _combine(Z):
    return pl.pallas_call(
        _combine_body,
        grid=(NB,),
        in_specs=[pl.BlockSpec((BM * NA, DM), lambda t: (t, 0))],
        out_specs=pl.BlockSpec((BM, DM), lambda t: (t, 0)),
        out_shape=jax.ShapeDtypeStruct((NT, DM), jnp.float32),
    )(Z)


def kernel(view0, view1, proj_W, proj_b, router_W, expert_keys, W1, b1, W2, b2):
    V = jnp.concatenate([view0, view1], axis=0)
    H, IDX, GV = _routing(V, proj_W, proj_b, router_W, expert_keys)
    dest, gk, be = _dispatch(IDX, GV)
    dest_flat = dest.reshape(NTOT)
    srcrow, gates = _scatter(dest_flat, gk.reshape(NTOT))
    Xs = _make_gather(MAXR, NV * NT)(srcrow, H)
    Ys = _ffn(be.reshape(MAXB + 1), Xs, W1, b1, W2, b2, gates)
    Z = _make_gather(NTOT, MAXR)(dest_flat, Ys)
    out = _combine(Z)
    return out.reshape(1, NT, DM)


# speedup vs baseline: 1.5573x; 1.2053x over previous
import functools

import jax
import jax.numpy as jnp
from jax import lax
from jax.experimental import pallas as pl
from jax.experimental.pallas import tpu as pltpu
from jax.experimental.pallas import tpu_sc as plsc

DM = 768
DF = 3072
NE = 8
NT = 2048
NV = 2
NA = 4
NTOT = NT * NA
BM = 256
MAXB = NTOT // BM + NE
MAXR = MAXB * BM
NB = NT // BM


def _routing_body(v0_ref, v1_ref, pw_ref, pb_ref, rw_ref, keys_ref,
                  h_ref, i_ref, g_ref):
    v = jnp.where(pl.program_id(0) == 0, v0_ref[0], v1_ref[0])
    h = jnp.dot(v, pw_ref[0], preferred_element_type=jnp.float32) + pb_ref[0]
    r = jnp.dot(h, rw_ref[0], preferred_element_type=jnp.float32)
    keys = keys_ref[...]
    d2_cols = []
    for e in range(NE):
        diff = r - keys[e:e + 1, :]
        d2_cols.append(jnp.sum(diff * diff, axis=1, keepdims=True))
    d2 = jnp.concatenate(d2_cols, axis=1)
    logits = -jnp.sqrt(d2 + 1e-12)
    iota = lax.broadcasted_iota(jnp.int32, (BM, NE), 1)
    m1 = jnp.max(logits, axis=1, keepdims=True)
    i1 = jnp.min(jnp.where(logits == m1, iota, NE), axis=1, keepdims=True)
    l2 = jnp.where(iota == i1, -1e30, logits)
    m2 = jnp.max(l2, axis=1, keepdims=True)
    i2 = jnp.min(jnp.where(l2 == m2, iota, NE), axis=1, keepdims=True)
    e2 = jnp.exp(m2 - m1)
    den = 1.0 + e2
    h_ref[...] = h
    i_ref[...] = jnp.concatenate([i1, i2], axis=1)
    g_ref[...] = jnp.concatenate([1.0 / den, e2 / den], axis=1)


def _routing(view0, view1, proj_W, proj_b, router_W, expert_keys):
    return pl.pallas_call(
        _routing_body,
        grid=(NV, NB),
        in_specs=[
            pl.BlockSpec((1, BM, DM), lambda v, t: (0, t, 0)),
            pl.BlockSpec((1, BM, DM), lambda v, t: (0, t, 0)),
            pl.BlockSpec((1, DM, DM), lambda v, t: (v, 0, 0)),
            pl.BlockSpec((1, 1, DM), lambda v, t: (v, 0, 0)),
            pl.BlockSpec((1, DM, NE), lambda v, t: (v, 0, 0)),
            pl.BlockSpec((NE, NE), lambda v, t: (0, 0)),
        ],
        out_specs=[
            pl.BlockSpec((BM, DM), lambda v, t: (v * NB + t, 0)),
            pl.BlockSpec((BM, 2), lambda v, t: (v * NB + t, 0)),
            pl.BlockSpec((BM, 2), lambda v, t: (v * NB + t, 0)),
        ],
        out_shape=[
            jax.ShapeDtypeStruct((NV * NT, DM), jnp.float32),
            jax.ShapeDtypeStruct((NV * NT, 2), jnp.int32),
            jax.ShapeDtypeStruct((NV * NT, 2), jnp.float32),
        ],
    )(view0, view1, proj_W, proj_b.reshape(NV, 1, DM), router_W, expert_keys)


def _dispatch_body(idx_ref, gv_ref, dest_ref, gk_ref, be_ref):
    ek = jnp.concatenate([idx_ref[:NT, :], idx_ref[NT:, :]], axis=1)
    gk = jnp.concatenate([gv_ref[:NT, :], gv_ref[NT:, :]], axis=1)
    iota_e = lax.broadcasted_iota(jnp.int32, (NT, NE), 1)

    cnt_row = jnp.zeros((NT, NE), jnp.float32)
    for c in range(NA):
        cnt_row = cnt_row + (ek[:, c:c + 1] == iota_e).astype(jnp.float32)

    ltri = (lax.broadcasted_iota(jnp.int32, (NT, NT), 0)
            > lax.broadcasted_iota(jnp.int32, (NT, NT), 1)).astype(jnp.float32)
    cbefore = jnp.dot(ltri, cnt_row, preferred_element_type=jnp.float32)

    cnt = jnp.sum(cnt_row, axis=0, keepdims=True)
    cp = jnp.floor((cnt + (BM - 1)) * (1.0 / BM)) * BM
    x = cp
    for s in (1, 2, 4):
        x = x + jnp.concatenate(
            [jnp.zeros((1, s), jnp.float32), x[:, :NE - s]], axis=1)
    pad_off = x - cp

    dest_cols = []
    for c in range(NA):
        sel = (ek[:, c:c + 1] == iota_e).astype(jnp.float32)
        base = jnp.sum(sel * (cbefore + pad_off), axis=1, keepdims=True)
        wr = jnp.zeros((NT, 1), jnp.float32)
        for cprev in range(c):
            wr = wr + (ek[:, cprev:cprev + 1] == ek[:, c:c + 1]).astype(jnp.float32)
        dest_cols.append(base + wr)
    dest_ref[...] = jnp.concatenate(dest_cols, axis=1).astype(jnp.int32)
    gk_ref[...] = gk

    ends = jnp.broadcast_to(pad_off + cp, (MAXB, NE))
    bio = lax.broadcasted_iota(jnp.int32, (MAXB, NE), 0).astype(jnp.float32) * BM
    be = jnp.sum((ends <= bio).astype(jnp.int32), axis=1, keepdims=True)
    nb_used = jnp.sum(cp * (1.0 / BM), axis=1, keepdims=True).astype(jnp.int32)
    be_ref[...] = jnp.concatenate([jnp.minimum(be, NE - 1), nb_used], axis=0)


def _dispatch(IDX, GV):
    return pl.pallas_call(
        _dispatch_body,
        out_shape=[
            jax.ShapeDtypeStruct((NT, NA), jnp.int32),
            jax.ShapeDtypeStruct((NT, NA), jnp.float32),
            jax.ShapeDtypeStruct((MAXB + 1, 1), jnp.int32),
        ],
    )(IDX, GV)


def _scatter_kernel(dest_hbm, gk_hbm, zi_hbm, zf_hbm, srcrow_hbm, gates_hbm,
                    dest_v, gk_v, sr_v, gt_v):
    wid = lax.axis_index("s") * 2 + lax.axis_index("c")

    @pl.when(wid == 0)
    def _():
        pltpu.sync_copy(dest_hbm, dest_v)
        pltpu.sync_copy(gk_hbm, gk_v)
        pltpu.sync_copy(zi_hbm, sr_v)
        pltpu.sync_copy(zf_hbm, gt_v)

        @plsc.parallel_loop(0, NTOT // 16, unroll=4)
        def _body(i):
            j0 = i * 16
            d = dest_v[pl.ds(j0, 16)]
            gt = gk_v[pl.ds(j0, 16)]
            j = lax.iota(jnp.int32, 16) + j0
            src = (j >> 2) + jnp.where((j & 3) >= 2, NT, 0)
            plsc.store_scatter(sr_v, [d], src)
            plsc.store_scatter(gt_v, [d], gt)

        pltpu.sync_copy(sr_v, srcrow_hbm)
        pltpu.sync_copy(gt_v, gates_hbm)


def _scatter(dest_flat, gk_flat):
    mesh = plsc.VectorSubcoreMesh(core_axis_name="c", subcore_axis_name="s")
    return pl.kernel(
        _scatter_kernel,
        mesh=mesh,
        compiler_params=pltpu.CompilerParams(needs_layout_passes=False),
        out_type=[
            jax.ShapeDtypeStruct((MAXR,), jnp.int32),
            jax.ShapeDtypeStruct((MAXR,), jnp.float32),
        ],
        scratch_types=[
            pltpu.VMEM((NTOT,), jnp.int32),
            pltpu.VMEM((NTOT,), jnp.float32),
            pltpu.VMEM((MAXR,), jnp.int32),
            pltpu.VMEM((MAXR,), jnp.float32),
        ],
    )(dest_flat, gk_flat,
      jnp.arange(MAXR, dtype=jnp.int32) % (NV * NT),
      jnp.zeros((MAXR,), jnp.float32))


def _make_gather(n_rows, n_src_rows):
    per_w = n_rows // 32
    ch = 64
    n_ch = per_w // ch
    mesh = plsc.VectorSubcoreMesh(core_axis_name="c", subcore_axis_name="s")

    def body(idx_hbm, src_hbm, out_hbm, idx_v, buf0, buf1, sg0, sg1, sw0, sw1):
        wid = lax.axis_index("s") * 2 + lax.axis_index("c")
        base = wid * per_w
        pltpu.sync_copy(idx_hbm.at[pl.ds(base, per_w)], idx_v)
        bufs = (buf0, buf1)
        sgs = (sg0, sg1)
        sws = (sw0, sw1)

        def start_gather(q):
            return pltpu.async_copy(
                src_hbm.at[idx_v.at[pl.ds(q * ch, ch)]], bufs[q % 2], sgs[q % 2])

        def start_write(q):
            return pltpu.async_copy(
                bufs[q % 2], out_hbm.at[pl.ds(base + q * ch, ch)], sws[q % 2])

        dg = {0: start_gather(0)}
        dw = {}
        for q in range(n_ch):
            if q + 1 < n_ch:
                if q >= 1:
                    dw[q - 1].wait()
                dg[q + 1] = start_gather(q + 1)
            dg[q].wait()
            dw[q] = start_write(q)
        dw[n_ch - 1].wait()
        if n_ch >= 2:
            dw[n_ch - 2].wait()

    return pl.kernel(
        body,
        mesh=mesh,
        out_type=jax.ShapeDtypeStruct((n_rows, DM), jnp.float32),
        scratch_types=[
            pltpu.VMEM((per_w,), jnp.int32),
            pltpu.VMEM((ch, DM), jnp.float32),
            pltpu.VMEM((ch, DM), jnp.float32),
            pltpu.SemaphoreType.DMA,
            pltpu.SemaphoreType.DMA,
            pltpu.SemaphoreType.DMA,
            pltpu.SemaphoreType.DMA,
        ],
    )


def _ffn_body(be_ref, xs_ref, w1_ref, b1_ref, w2_ref, b2_ref, g_ref, out_ref):
    b = pl.program_id(0)

    @pl.when(b < be_ref[MAXB])
    def _():
        x = xs_ref[...].astype(jnp.bfloat16)
        mid = jnp.dot(x, w1_ref[0].astype(jnp.bfloat16),
                      preferred_element_type=jnp.float32) + b1_ref[0]
        mid = 0.5 * mid * (1.0 + lax.erf(mid * 0.7071067811865476))
        y = jnp.dot(mid.astype(jnp.bfloat16), w2_ref[0].astype(jnp.bfloat16),
                    preferred_element_type=jnp.float32) + b2_ref[0]
        out_ref[...] = g_ref[...] * y

    @pl.when(b >= be_ref[MAXB])
    def _():
        out_ref[...] = jnp.zeros_like(out_ref)


def _ffn(be, Xs, W1, b1, W2, b2, gates):
    grid_spec = pltpu.PrefetchScalarGridSpec(
        num_scalar_prefetch=1,
        grid=(MAXB,),
        in_specs=[
            pl.BlockSpec((BM, DM), lambda b, be_ref: (b, 0)),
            pl.BlockSpec((1, DM, DF), lambda b, be_ref: (be_ref[b], 0, 0)),
            pl.BlockSpec((1, 1, DF), lambda b, be_ref: (be_ref[b], 0, 0)),
            pl.BlockSpec((1, DF, DM), lambda b, be_ref: (be_ref[b], 0, 0)),
            pl.BlockSpec((1, 1, DM), lambda b, be_ref: (be_ref[b], 0, 0)),
            pl.BlockSpec((BM, 1), lambda b, be_ref: (b, 0)),
        ],
        out_specs=pl.BlockSpec((BM, DM), lambda b, be_ref: (b, 0)),
    )
    return pl.pallas_call(
        _ffn_body,
        grid_spec=grid_spec,
        out_shape=jax.ShapeDtypeStruct((MAXR, DM), jnp.float32),
    )(be, Xs, W1, b1.reshape(NE, 1, DF), W2, b2.reshape(NE, 1, DM),
      gates.reshape(MAXR, 1))


def _combine_body(z_ref, out_ref):
    x = z_ref[...].reshape(BM, NA, DM)
    out_ref[...] = jnp.sum(x, axis=1)


def _combine(Z):
    return pl.pallas_call(
        _combine_body,
        grid=(NB,),
        in_specs=[pl.BlockSpec((BM * NA, DM), lambda t: (t, 0))],
        out_specs=pl.BlockSpec((BM, DM), lambda t: (t, 0)),
        out_shape=jax.ShapeDtypeStruct((NT, DM), jnp.float32),
    )(Z)


def kernel(view0, view1, proj_W, proj_b, router_W, expert_keys, W1, b1, W2, b2):
    H, IDX, GV = _routing(view0, view1, proj_W, proj_b, router_W, expert_keys)
    dest, gk, be = _dispatch(IDX, GV)
    dest_flat = dest.reshape(NTOT)
    srcrow, gates = _scatter(dest_flat, gk.reshape(NTOT))
    Xs = _make_gather(MAXR, NV * NT)(srcrow, H)
    Ys = _ffn(be.reshape(MAXB + 1), Xs, W1, b1, W2, b2, gates)
    Z = _make_gather(NTOT, MAXR)(dest_flat, Ys)
    out = _combine(Z)
    return out.reshape(1, NT, DM)


# speedup vs baseline: 1.5721x; 1.0095x over previous
import functools

import jax
import jax.numpy as jnp
from jax import lax
from jax.experimental import pallas as pl
from jax.experimental.pallas import tpu as pltpu
from jax.experimental.pallas import tpu_sc as plsc

DM = 768
DF = 3072
NE = 8
NT = 2048
NV = 2
NA = 4
NTOT = NT * NA
BM = 256
MAXB = NTOT // BM + NE
MAXR = MAXB * BM
NB = NT // BM
BR = 512
NBR = NT // BR


def _routing_body(v0_ref, v1_ref, pw_ref, pb_ref, rw_ref, keys_ref,
                  h_ref, i_ref, g_ref):
    v = jnp.where(pl.program_id(0) == 0, v0_ref[0], v1_ref[0])
    h = jnp.dot(v, pw_ref[0], preferred_element_type=jnp.float32) + pb_ref[0]
    r = jnp.dot(h, rw_ref[0], preferred_element_type=jnp.float32)
    keys = keys_ref[...]
    d2_cols = []
    for e in range(NE):
        diff = r - keys[e:e + 1, :]
        d2_cols.append(jnp.sum(diff * diff, axis=1, keepdims=True))
    d2 = jnp.concatenate(d2_cols, axis=1)
    logits = -jnp.sqrt(d2 + 1e-12)
    iota = lax.broadcasted_iota(jnp.int32, (BR, NE), 1)
    m1 = jnp.max(logits, axis=1, keepdims=True)
    i1 = jnp.min(jnp.where(logits == m1, iota, NE), axis=1, keepdims=True)
    l2 = jnp.where(iota == i1, -1e30, logits)
    m2 = jnp.max(l2, axis=1, keepdims=True)
    i2 = jnp.min(jnp.where(l2 == m2, iota, NE), axis=1, keepdims=True)
    e2 = jnp.exp(m2 - m1)
    den = 1.0 + e2
    h_ref[...] = h
    i_ref[...] = jnp.concatenate([i1, i2], axis=1)
    g_ref[...] = jnp.concatenate([1.0 / den, e2 / den], axis=1)


def _routing(view0, view1, proj_W, proj_b, router_W, expert_keys):
    return pl.pallas_call(
        _routing_body,
        grid=(NV, NBR),
        in_specs=[
            pl.BlockSpec((1, BR, DM), lambda v, t: (0, t, 0)),
            pl.BlockSpec((1, BR, DM), lambda v, t: (0, t, 0)),
            pl.BlockSpec((1, DM, DM), lambda v, t: (v, 0, 0)),
            pl.BlockSpec((1, 1, DM), lambda v, t: (v, 0, 0)),
            pl.BlockSpec((1, DM, NE), lambda v, t: (v, 0, 0)),
            pl.BlockSpec((NE, NE), lambda v, t: (0, 0)),
        ],
        out_specs=[
            pl.BlockSpec((BR, DM), lambda v, t: (v * NBR + t, 0)),
            pl.BlockSpec((BR, 2), lambda v, t: (v * NBR + t, 0)),
            pl.BlockSpec((BR, 2), lambda v, t: (v * NBR + t, 0)),
        ],
        out_shape=[
            jax.ShapeDtypeStruct((NV * NT, DM), jnp.float32),
            jax.ShapeDtypeStruct((NV * NT, 2), jnp.int32),
            jax.ShapeDtypeStruct((NV * NT, 2), jnp.float32),
        ],
    )(view0, view1, proj_W, proj_b.reshape(NV, 1, DM), router_W, expert_keys)


def _dispatch_body(idx_ref, gv_ref, dest_ref, gk_ref, be_ref):
    ek = jnp.concatenate([idx_ref[:NT, :], idx_ref[NT:, :]], axis=1)
    gk = jnp.concatenate([gv_ref[:NT, :], gv_ref[NT:, :]], axis=1)
    iota_e = lax.broadcasted_iota(jnp.int32, (NT, NE), 1)

    cnt_row = jnp.zeros((NT, NE), jnp.float32)
    for c in range(NA):
        cnt_row = cnt_row + (ek[:, c:c + 1] == iota_e).astype(jnp.float32)

    ltri = (lax.broadcasted_iota(jnp.int32, (NT, NT), 0)
            > lax.broadcasted_iota(jnp.int32, (NT, NT), 1)).astype(jnp.float32)
    cbefore = jnp.dot(ltri, cnt_row, preferred_element_type=jnp.float32)

    cnt = jnp.sum(cnt_row, axis=0, keepdims=True)
    cp = jnp.floor((cnt + (BM - 1)) * (1.0 / BM)) * BM
    x = cp
    for s in (1, 2, 4):
        x = x + jnp.concatenate(
            [jnp.zeros((1, s), jnp.float32), x[:, :NE - s]], axis=1)
    pad_off = x - cp

    dest_cols = []
    for c in range(NA):
        sel = (ek[:, c:c + 1] == iota_e).astype(jnp.float32)
        base = jnp.sum(sel * (cbefore + pad_off), axis=1, keepdims=True)
        wr = jnp.zeros((NT, 1), jnp.float32)
        for cprev in range(c):
            wr = wr + (ek[:, cprev:cprev + 1] == ek[:, c:c + 1]).astype(jnp.float32)
        dest_cols.append(base + wr)
    dest_ref[...] = jnp.concatenate(dest_cols, axis=1).astype(jnp.int32)
    gk_ref[...] = gk

    ends = jnp.broadcast_to(pad_off + cp, (MAXB, NE))
    bio = lax.broadcasted_iota(jnp.int32, (MAXB, NE), 0).astype(jnp.float32) * BM
    be = jnp.sum((ends <= bio).astype(jnp.int32), axis=1, keepdims=True)
    nb_used = jnp.sum(cp * (1.0 / BM), axis=1, keepdims=True).astype(jnp.int32)
    be_ref[...] = jnp.concatenate([jnp.minimum(be, NE - 1), nb_used], axis=0)


def _dispatch(IDX, GV):
    return pl.pallas_call(
        _dispatch_body,
        out_shape=[
            jax.ShapeDtypeStruct((NT, NA), jnp.int32),
            jax.ShapeDtypeStruct((NT, NA), jnp.float32),
            jax.ShapeDtypeStruct((MAXB + 1, 1), jnp.int32),
        ],
    )(IDX, GV)


def _scatter_kernel(dest_hbm, gk_hbm, zi_hbm, zf_hbm, srcrow_hbm, gates_hbm,
                    dest_v, gk_v, sr_v, gt_v):
    wid = lax.axis_index("s") * 2 + lax.axis_index("c")

    @pl.when(wid == 0)
    def _():
        pltpu.sync_copy(dest_hbm, dest_v)
        pltpu.sync_copy(gk_hbm, gk_v)
        pltpu.sync_copy(zi_hbm, sr_v)
        pltpu.sync_copy(zf_hbm, gt_v)

        @plsc.parallel_loop(0, NTOT // 16, unroll=4)
        def _body(i):
            j0 = i * 16
            d = dest_v[pl.ds(j0, 16)]
            gt = gk_v[pl.ds(j0, 16)]
            j = lax.iota(jnp.int32, 16) + j0
            src = (j >> 2) + jnp.where((j & 3) >= 2, NT, 0)
            plsc.store_scatter(sr_v, [d], src)
            plsc.store_scatter(gt_v, [d], gt)

        pltpu.sync_copy(sr_v, srcrow_hbm)
        pltpu.sync_copy(gt_v, gates_hbm)


def _scatter(dest_flat, gk_flat):
    mesh = plsc.VectorSubcoreMesh(core_axis_name="c", subcore_axis_name="s")
    return pl.kernel(
        _scatter_kernel,
        mesh=mesh,
        compiler_params=pltpu.CompilerParams(needs_layout_passes=False),
        out_type=[
            jax.ShapeDtypeStruct((MAXR,), jnp.int32),
            jax.ShapeDtypeStruct((MAXR,), jnp.float32),
        ],
        scratch_types=[
            pltpu.VMEM((NTOT,), jnp.int32),
            pltpu.VMEM((NTOT,), jnp.float32),
            pltpu.VMEM((MAXR,), jnp.int32),
            pltpu.VMEM((MAXR,), jnp.float32),
        ],
    )(dest_flat, gk_flat,
      jnp.arange(MAXR, dtype=jnp.int32) % (NV * NT),
      jnp.zeros((MAXR,), jnp.float32))


def _make_gather(n_rows, n_src_rows):
    per_w = n_rows // 32
    ch = 64
    n_ch = per_w // ch
    mesh = plsc.VectorSubcoreMesh(core_axis_name="c", subcore_axis_name="s")

    def body(idx_hbm, src_hbm, out_hbm, idx_v, buf0, buf1, sg0, sg1, sw0, sw1):
        wid = lax.axis_index("s") * 2 + lax.axis_index("c")
        base = wid * per_w
        pltpu.sync_copy(idx_hbm.at[pl.ds(base, per_w)], idx_v)
        bufs = (buf0, buf1)
        sgs = (sg0, sg1)
        sws = (sw0, sw1)

        def start_gather(q):
            return pltpu.async_copy(
                src_hbm.at[idx_v.at[pl.ds(q * ch, ch)]], bufs[q % 2], sgs[q % 2])

        def start_write(q):
            return pltpu.async_copy(
                bufs[q % 2], out_hbm.at[pl.ds(base + q * ch, ch)], sws[q % 2])

        dg = {0: start_gather(0)}
        dw = {}
        for q in range(n_ch):
            if q + 1 < n_ch:
                if q >= 1:
                    dw[q - 1].wait()
                dg[q + 1] = start_gather(q + 1)
            dg[q].wait()
            dw[q] = start_write(q)
        dw[n_ch - 1].wait()
        if n_ch >= 2:
            dw[n_ch - 2].wait()

    return pl.kernel(
        body,
        mesh=mesh,
        out_type=jax.ShapeDtypeStruct((n_rows, DM), jnp.float32),
        scratch_types=[
            pltpu.VMEM((per_w,), jnp.int32),
            pltpu.VMEM((ch, DM), jnp.float32),
            pltpu.VMEM((ch, DM), jnp.float32),
            pltpu.SemaphoreType.DMA,
            pltpu.SemaphoreType.DMA,
            pltpu.SemaphoreType.DMA,
            pltpu.SemaphoreType.DMA,
        ],
    )


def _ffn_body(be_ref, xs_ref, w1_ref, b1_ref, w2_ref, b2_ref, g_ref, out_ref,
              w1b_ref, w2b_ref):
    b = pl.program_id(0)
    live = b < be_ref[MAXB]
    new_expert = jnp.logical_or(b == 0,
                                be_ref[b] != be_ref[jnp.maximum(b - 1, 0)])

    @pl.when(jnp.logical_and(live, new_expert))
    def _():
        w1b_ref[...] = w1_ref[0].astype(jnp.bfloat16)
        w2b_ref[...] = w2_ref[0].astype(jnp.bfloat16)

    @pl.when(live)
    def _():
        x = xs_ref[...].astype(jnp.bfloat16)
        mid = jnp.dot(x, w1b_ref[...],
                      preferred_element_type=jnp.float32) + b1_ref[0]
        mid = 0.5 * mid * (1.0 + lax.erf(mid * 0.7071067811865476))
        y = jnp.dot(mid.astype(jnp.bfloat16), w2b_ref[...],
                    preferred_element_type=jnp.float32) + b2_ref[0]
        out_ref[...] = g_ref[...] * y

    @pl.when(jnp.logical_not(live))
    def _():
        out_ref[...] = jnp.zeros_like(out_ref)


def _ffn(be, Xs, W1, b1, W2, b2, gates):
    grid_spec = pltpu.PrefetchScalarGridSpec(
        num_scalar_prefetch=1,
        grid=(MAXB,),
        in_specs=[
            pl.BlockSpec((BM, DM), lambda b, be_ref: (b, 0)),
            pl.BlockSpec((1, DM, DF), lambda b, be_ref: (be_ref[b], 0, 0)),
            pl.BlockSpec((1, 1, DF), lambda b, be_ref: (be_ref[b], 0, 0)),
            pl.BlockSpec((1, DF, DM), lambda b, be_ref: (be_ref[b], 0, 0)),
            pl.BlockSpec((1, 1, DM), lambda b, be_ref: (be_ref[b], 0, 0)),
            pl.BlockSpec((BM, 1), lambda b, be_ref: (b, 0)),
        ],
        out_specs=pl.BlockSpec((BM, DM), lambda b, be_ref: (b, 0)),
        scratch_shapes=[
            pltpu.VMEM((DM, DF), jnp.bfloat16),
            pltpu.VMEM((DF, DM), jnp.bfloat16),
        ],
    )
    return pl.pallas_call(
        _ffn_body,
        grid_spec=grid_spec,
        out_shape=jax.ShapeDtypeStruct((MAXR, DM), jnp.float32),
    )(be, Xs, W1, b1.reshape(NE, 1, DF), W2, b2.reshape(NE, 1, DM),
      gates.reshape(MAXR, 1))


def _combine_body(z_ref, out_ref):
    x = z_ref[...].reshape(BM, NA, DM)
    out_ref[...] = jnp.sum(x, axis=1)


def _combine(Z):
    return pl.pallas_call(
        _combine_body,
        grid=(NB,),
        in_specs=[pl.BlockSpec((BM * NA, DM), lambda t: (t, 0))],
        out_specs=pl.BlockSpec((BM, DM), lambda t: (t, 0)),
        out_shape=jax.ShapeDtypeStruct((NT, DM), jnp.float32),
    )(Z)


def kernel(view0, view1, proj_W, proj_b, router_W, expert_keys, W1, b1, W2, b2):
    H, IDX, GV = _routing(view0, view1, proj_W, proj_b, router_W, expert_keys)
    dest, gk, be = _dispatch(IDX, GV)
    dest_flat = dest.reshape(NTOT)
    srcrow, gates = _scatter(dest_flat, gk.reshape(NTOT))
    Xs = _make_gather(MAXR, NV * NT)(srcrow, H)
    Ys = _ffn(be.reshape(MAXB + 1), Xs, W1, b1, W2, b2, gates)
    Z = _make_gather(NTOT, MAXR)(dest_flat, Ys)
    out = _combine(Z)
    return out.reshape(1, NT, DM)


# speedup vs baseline: 1.6704x; 1.0626x over previous
import functools

import jax
import jax.numpy as jnp
from jax import lax
from jax.experimental import pallas as pl
from jax.experimental.pallas import tpu as pltpu
from jax.experimental.pallas import tpu_sc as plsc

DM = 768
DF = 3072
NE = 8
NT = 2048
NV = 2
NA = 4
NTOT = NT * NA
BM = 256
MAXB = NTOT // BM + NE
MAXR = MAXB * BM
NB = NT // BM
BR = 512
NBR = NT // BR


def _routing_body(v0_ref, v1_ref, pw_ref, pb_ref, rw_ref, keys_ref,
                  h_ref, i_ref, g_ref):
    v = jnp.where(pl.program_id(0) == 0, v0_ref[0], v1_ref[0])
    h = jnp.dot(v, pw_ref[0], preferred_element_type=jnp.float32) + pb_ref[0]
    r = jnp.dot(h, rw_ref[0], preferred_element_type=jnp.float32)
    keys = keys_ref[...]
    d2_cols = []
    for e in range(NE):
        diff = r - keys[e:e + 1, :]
        d2_cols.append(jnp.sum(diff * diff, axis=1, keepdims=True))
    d2 = jnp.concatenate(d2_cols, axis=1)
    logits = -jnp.sqrt(d2 + 1e-12)
    iota = lax.broadcasted_iota(jnp.int32, (BR, NE), 1)
    m1 = jnp.max(logits, axis=1, keepdims=True)
    i1 = jnp.min(jnp.where(logits == m1, iota, NE), axis=1, keepdims=True)
    l2 = jnp.where(iota == i1, -1e30, logits)
    m2 = jnp.max(l2, axis=1, keepdims=True)
    i2 = jnp.min(jnp.where(l2 == m2, iota, NE), axis=1, keepdims=True)
    e2 = jnp.exp(m2 - m1)
    den = 1.0 + e2
    h_ref[...] = h
    i_ref[...] = jnp.concatenate([i1, i2], axis=1)
    g_ref[...] = jnp.concatenate([1.0 / den, e2 / den], axis=1)


def _routing(view0, view1, proj_W, proj_b, router_W, expert_keys):
    return pl.pallas_call(
        _routing_body,
        grid=(NV, NBR),
        in_specs=[
            pl.BlockSpec((1, BR, DM), lambda v, t: (0, t, 0)),
            pl.BlockSpec((1, BR, DM), lambda v, t: (0, t, 0)),
            pl.BlockSpec((1, DM, DM), lambda v, t: (v, 0, 0)),
            pl.BlockSpec((1, 1, DM), lambda v, t: (v, 0, 0)),
            pl.BlockSpec((1, DM, NE), lambda v, t: (v, 0, 0)),
            pl.BlockSpec((NE, NE), lambda v, t: (0, 0)),
        ],
        out_specs=[
            pl.BlockSpec((BR, DM), lambda v, t: (v * NBR + t, 0)),
            pl.BlockSpec((BR, 2), lambda v, t: (v * NBR + t, 0)),
            pl.BlockSpec((BR, 2), lambda v, t: (v * NBR + t, 0)),
        ],
        out_shape=[
            jax.ShapeDtypeStruct((NV * NT, DM), jnp.float32),
            jax.ShapeDtypeStruct((NV * NT, 2), jnp.int32),
            jax.ShapeDtypeStruct((NV * NT, 2), jnp.float32),
        ],
    )(view0, view1, proj_W, proj_b.reshape(NV, 1, DM), router_W, expert_keys)


def _dispatch_body(idx_ref, gv_ref, dest_ref, gk_ref, be_ref):
    ek = jnp.concatenate([idx_ref[:NT, :], idx_ref[NT:, :]], axis=1)
    gk = jnp.concatenate([gv_ref[:NT, :], gv_ref[NT:, :]], axis=1)
    iota_e = lax.broadcasted_iota(jnp.int32, (NT, NE), 1)

    cnt_row = jnp.zeros((NT, NE), jnp.float32)
    for c in range(NA):
        cnt_row = cnt_row + (ek[:, c:c + 1] == iota_e).astype(jnp.float32)

    ltri = (lax.broadcasted_iota(jnp.int32, (NT, NT), 0)
            > lax.broadcasted_iota(jnp.int32, (NT, NT), 1)).astype(jnp.float32)
    cbefore = jnp.dot(ltri, cnt_row, preferred_element_type=jnp.float32)

    cnt = jnp.sum(cnt_row, axis=0, keepdims=True)
    cp = jnp.floor((cnt + (BM - 1)) * (1.0 / BM)) * BM
    x = cp
    for s in (1, 2, 4):
        x = x + jnp.concatenate(
            [jnp.zeros((1, s), jnp.float32), x[:, :NE - s]], axis=1)
    pad_off = x - cp

    dest_cols = []
    for c in range(NA):
        sel = (ek[:, c:c + 1] == iota_e).astype(jnp.float32)
        base = jnp.sum(sel * (cbefore + pad_off), axis=1, keepdims=True)
        wr = jnp.zeros((NT, 1), jnp.float32)
        for cprev in range(c):
            wr = wr + (ek[:, cprev:cprev + 1] == ek[:, c:c + 1]).astype(jnp.float32)
        dest_cols.append(base + wr)
    dest_ref[...] = jnp.concatenate(dest_cols, axis=1).astype(jnp.int32)
    gk_ref[...] = gk

    ends = jnp.broadcast_to(pad_off + cp, (MAXB, NE))
    bio = lax.broadcasted_iota(jnp.int32, (MAXB, NE), 0).astype(jnp.float32) * BM
    be = jnp.sum((ends <= bio).astype(jnp.int32), axis=1, keepdims=True)
    nb_used = jnp.sum(cp * (1.0 / BM), axis=1, keepdims=True).astype(jnp.int32)
    be_ref[...] = jnp.concatenate([jnp.minimum(be, NE - 1), nb_used], axis=0)


def _dispatch(IDX, GV):
    return pl.pallas_call(
        _dispatch_body,
        out_shape=[
            jax.ShapeDtypeStruct((NT, NA), jnp.int32),
            jax.ShapeDtypeStruct((NT, NA), jnp.float32),
            jax.ShapeDtypeStruct((MAXB + 1, 1), jnp.int32),
        ],
    )(IDX, GV)


def _scatter_kernel(dest_hbm, gk_hbm, zi_hbm, zf_hbm, srcrow_hbm, gates_hbm,
                    dest_v, gk_v, sr_v, gt_v):
    wid = lax.axis_index("s") * 2 + lax.axis_index("c")

    @pl.when(wid == 0)
    def _():
        pltpu.sync_copy(dest_hbm, dest_v)
        pltpu.sync_copy(gk_hbm, gk_v)
        pltpu.sync_copy(zi_hbm, sr_v)
        pltpu.sync_copy(zf_hbm, gt_v)

        @plsc.parallel_loop(0, NTOT // 16, unroll=4)
        def _body(i):
            j0 = i * 16
            d = dest_v[pl.ds(j0, 16)]
            gt = gk_v[pl.ds(j0, 16)]
            j = lax.iota(jnp.int32, 16) + j0
            src = (j >> 2) + jnp.where((j & 3) >= 2, NT, 0)
            plsc.store_scatter(sr_v, [d], src)
            plsc.store_scatter(gt_v, [d], gt)

        pltpu.sync_copy(sr_v, srcrow_hbm)
        pltpu.sync_copy(gt_v, gates_hbm)


def _scatter(dest_flat, gk_flat):
    mesh = plsc.VectorSubcoreMesh(core_axis_name="c", subcore_axis_name="s")
    return pl.kernel(
        _scatter_kernel,
        mesh=mesh,
        compiler_params=pltpu.CompilerParams(needs_layout_passes=False),
        out_type=[
            jax.ShapeDtypeStruct((MAXR,), jnp.int32),
            jax.ShapeDtypeStruct((MAXR,), jnp.float32),
        ],
        scratch_types=[
            pltpu.VMEM((NTOT,), jnp.int32),
            pltpu.VMEM((NTOT,), jnp.float32),
            pltpu.VMEM((MAXR,), jnp.int32),
            pltpu.VMEM((MAXR,), jnp.float32),
        ],
    )(dest_flat, gk_flat,
      jnp.arange(MAXR, dtype=jnp.int32) % (NV * NT),
      jnp.zeros((MAXR,), jnp.float32))


def _make_gather(n_rows, n_src_rows):
    per_w = n_rows // 32
    ch = 64
    n_ch = per_w // ch
    mesh = plsc.VectorSubcoreMesh(core_axis_name="c", subcore_axis_name="s")

    def body(idx_hbm, src_hbm, out_hbm, idx_v, buf0, buf1, sg0, sg1, sw0, sw1):
        wid = lax.axis_index("s") * 2 + lax.axis_index("c")
        base = wid * per_w
        pltpu.sync_copy(idx_hbm.at[pl.ds(base, per_w)], idx_v)
        bufs = (buf0, buf1)
        sgs = (sg0, sg1)
        sws = (sw0, sw1)

        def start_gather(q):
            return pltpu.async_copy(
                src_hbm.at[idx_v.at[pl.ds(q * ch, ch)]], bufs[q % 2], sgs[q % 2])

        def start_write(q):
            return pltpu.async_copy(
                bufs[q % 2], out_hbm.at[pl.ds(base + q * ch, ch)], sws[q % 2])

        dg = {0: start_gather(0)}
        dw = {}
        for q in range(n_ch):
            if q + 1 < n_ch:
                if q >= 1:
                    dw[q - 1].wait()
                dg[q + 1] = start_gather(q + 1)
            dg[q].wait()
            dw[q] = start_write(q)
        dw[n_ch - 1].wait()
        if n_ch >= 2:
            dw[n_ch - 2].wait()

    return pl.kernel(
        body,
        mesh=mesh,
        out_type=jax.ShapeDtypeStruct((n_rows, DM), jnp.float32),
        scratch_types=[
            pltpu.VMEM((per_w,), jnp.int32),
            pltpu.VMEM((ch, DM), jnp.float32),
            pltpu.VMEM((ch, DM), jnp.float32),
            pltpu.SemaphoreType.DMA,
            pltpu.SemaphoreType.DMA,
            pltpu.SemaphoreType.DMA,
            pltpu.SemaphoreType.DMA,
        ],
    )


def _ffn_body(be_ref, xs_ref, w1_ref, b1_ref, w2_ref, b2_ref, g_ref, out_ref,
              w1b_ref, w2b_ref):
    b = pl.program_id(0)
    live = b < be_ref[MAXB]
    new_expert = jnp.logical_or(b == 0,
                                be_ref[b] != be_ref[jnp.maximum(b - 1, 0)])

    @pl.when(jnp.logical_and(live, new_expert))
    def _():
        w1b_ref[...] = w1_ref[0].astype(jnp.bfloat16)
        w2b_ref[...] = w2_ref[0].astype(jnp.bfloat16)

    @pl.when(live)
    def _():
        x = xs_ref[...].astype(jnp.bfloat16)
        mid = jnp.dot(x, w1b_ref[...],
                      preferred_element_type=jnp.float32) + b1_ref[0]
        mid = 0.5 * mid * (1.0 + lax.erf(mid * 0.7071067811865476))
        y = jnp.dot(mid.astype(jnp.bfloat16), w2b_ref[...],
                    preferred_element_type=jnp.float32) + b2_ref[0]
        out_ref[...] = g_ref[...] * y

    @pl.when(jnp.logical_not(live))
    def _():
        out_ref[...] = jnp.zeros_like(out_ref)


def _ffn(be, Xs, W1, b1, W2, b2, gates):
    grid_spec = pltpu.PrefetchScalarGridSpec(
        num_scalar_prefetch=1,
        grid=(MAXB,),
        in_specs=[
            pl.BlockSpec((BM, DM), lambda b, be_ref: (b, 0)),
            pl.BlockSpec((1, DM, DF), lambda b, be_ref: (be_ref[b], 0, 0)),
            pl.BlockSpec((1, 1, DF), lambda b, be_ref: (be_ref[b], 0, 0)),
            pl.BlockSpec((1, DF, DM), lambda b, be_ref: (be_ref[b], 0, 0)),
            pl.BlockSpec((1, 1, DM), lambda b, be_ref: (be_ref[b], 0, 0)),
            pl.BlockSpec((BM, 1), lambda b, be_ref: (b, 0)),
        ],
        out_specs=pl.BlockSpec((BM, DM), lambda b, be_ref: (b, 0)),
        scratch_shapes=[
            pltpu.VMEM((DM, DF), jnp.bfloat16),
            pltpu.VMEM((DF, DM), jnp.bfloat16),
        ],
    )
    return pl.pallas_call(
        _ffn_body,
        grid_spec=grid_spec,
        out_shape=jax.ShapeDtypeStruct((MAXR, DM), jnp.float32),
    )(be, Xs, W1, b1.reshape(NE, 1, DF), W2, b2.reshape(NE, 1, DM),
      gates.reshape(MAXR, 1))


def _combine_sc(dest_flat, Ys):
    tpw = NT // 32
    tch = 16
    rch = tch * NA
    n_ch = tpw // tch
    mesh = plsc.VectorSubcoreMesh(core_axis_name="c", subcore_axis_name="s")

    def body(idx_hbm, ys_hbm, out_hbm, idx_v, buf0, buf1, ov0, ov1,
             sg0, sg1, sw0, sw1):
        wid = lax.axis_index("s") * 2 + lax.axis_index("c")
        t0 = wid * tpw
        pltpu.sync_copy(idx_hbm.at[pl.ds(t0 * NA, tpw * NA)], idx_v)
        bufs = (buf0, buf1)
        ovs = (ov0, ov1)
        sgs = (sg0, sg1)
        sws = (sw0, sw1)

        def start_gather(q):
            return pltpu.async_copy(
                ys_hbm.at[idx_v.at[pl.ds(q * rch, rch)]], bufs[q % 2], sgs[q % 2])

        dg = {0: start_gather(0)}
        dw = {}
        for q in range(n_ch):
            if q + 1 < n_ch:
                dg[q + 1] = start_gather(q + 1)
            dg[q].wait()
            if q >= 2:
                dw[q - 2].wait()
            buf = bufs[q % 2]
            ov = ovs[q % 2]
            for tt in range(tch):
                @plsc.parallel_loop(0, DM // 16, unroll=4)
                def _add(l):
                    sl = pl.ds(l * 16, 16)
                    ov[tt, sl] = (buf[NA * tt, sl] + buf[NA * tt + 1, sl]
                                  + buf[NA * tt + 2, sl] + buf[NA * tt + 3, sl])
            dw[q] = pltpu.async_copy(
                ov, out_hbm.at[pl.ds(t0 + q * tch, tch)], sws[q % 2])
        for q in (n_ch - 2, n_ch - 1):
            dw[q].wait()

    return pl.kernel(
        body,
        mesh=mesh,
        compiler_params=pltpu.CompilerParams(needs_layout_passes=False),
        out_type=jax.ShapeDtypeStruct((NT, DM), jnp.float32),
        scratch_types=[
            pltpu.VMEM((tpw * NA,), jnp.int32),
            pltpu.VMEM((rch, DM), jnp.float32),
            pltpu.VMEM((rch, DM), jnp.float32),
            pltpu.VMEM((tch, DM), jnp.float32),
            pltpu.VMEM((tch, DM), jnp.float32),
            pltpu.SemaphoreType.DMA,
            pltpu.SemaphoreType.DMA,
            pltpu.SemaphoreType.DMA,
            pltpu.SemaphoreType.DMA,
        ],
    )(dest_flat, Ys)


def kernel(view0, view1, proj_W, proj_b, router_W, expert_keys, W1, b1, W2, b2):
    H, IDX, GV = _routing(view0, view1, proj_W, proj_b, router_W, expert_keys)
    dest, gk, be = _dispatch(IDX, GV)
    dest_flat = dest.reshape(NTOT)
    srcrow, gates = _scatter(dest_flat, gk.reshape(NTOT))
    Xs = _make_gather(MAXR, NV * NT)(srcrow, H)
    Ys = _ffn(be.reshape(MAXB + 1), Xs, W1, b1, W2, b2, gates)
    out = _combine_sc(dest_flat, Ys)
    return out.reshape(1, NT, DM)


# speedup vs baseline: 1.7503x; 1.0478x over previous
import functools

import jax
import jax.numpy as jnp
from jax import lax
from jax.experimental import pallas as pl
from jax.experimental.pallas import tpu as pltpu
from jax.experimental.pallas import tpu_sc as plsc

DM = 768
DF = 3072
NE = 8
NT = 2048
NV = 2
NA = 4
NTOT = NT * NA
BM = 512
MAXB = NTOT // BM + NE
MAXR = MAXB * BM
NB = NT // BM
BR = 512
NBR = NT // BR


def _routing_body(v0_ref, v1_ref, pw_ref, pb_ref, rw_ref, keys_ref,
                  h_ref, i_ref, g_ref):
    v = jnp.where(pl.program_id(0) == 0, v0_ref[0], v1_ref[0])
    h = jnp.dot(v, pw_ref[0], preferred_element_type=jnp.float32) + pb_ref[0]
    r = jnp.dot(h, rw_ref[0], preferred_element_type=jnp.float32)
    keys = keys_ref[...]
    d2_cols = []
    for e in range(NE):
        diff = r - keys[e:e + 1, :]
        d2_cols.append(jnp.sum(diff * diff, axis=1, keepdims=True))
    d2 = jnp.concatenate(d2_cols, axis=1)
    logits = -jnp.sqrt(d2 + 1e-12)
    iota = lax.broadcasted_iota(jnp.int32, (BR, NE), 1)
    m1 = jnp.max(logits, axis=1, keepdims=True)
    i1 = jnp.min(jnp.where(logits == m1, iota, NE), axis=1, keepdims=True)
    l2 = jnp.where(iota == i1, -1e30, logits)
    m2 = jnp.max(l2, axis=1, keepdims=True)
    i2 = jnp.min(jnp.where(l2 == m2, iota, NE), axis=1, keepdims=True)
    e2 = jnp.exp(m2 - m1)
    den = 1.0 + e2
    h_ref[...] = h
    i_ref[...] = jnp.concatenate([i1, i2], axis=1)
    g_ref[...] = jnp.concatenate([1.0 / den, e2 / den], axis=1)


def _routing(view0, view1, proj_W, proj_b, router_W, expert_keys):
    return pl.pallas_call(
        _routing_body,
        grid=(NV, NBR),
        in_specs=[
            pl.BlockSpec((1, BR, DM), lambda v, t: (0, t, 0)),
            pl.BlockSpec((1, BR, DM), lambda v, t: (0, t, 0)),
            pl.BlockSpec((1, DM, DM), lambda v, t: (v, 0, 0)),
            pl.BlockSpec((1, 1, DM), lambda v, t: (v, 0, 0)),
            pl.BlockSpec((1, DM, NE), lambda v, t: (v, 0, 0)),
            pl.BlockSpec((NE, NE), lambda v, t: (0, 0)),
        ],
        out_specs=[
            pl.BlockSpec((BR, DM), lambda v, t: (v * NBR + t, 0)),
            pl.BlockSpec((BR, 2), lambda v, t: (v * NBR + t, 0)),
            pl.BlockSpec((BR, 2), lambda v, t: (v * NBR + t, 0)),
        ],
        out_shape=[
            jax.ShapeDtypeStruct((NV * NT, DM), jnp.float32),
            jax.ShapeDtypeStruct((NV * NT, 2), jnp.int32),
            jax.ShapeDtypeStruct((NV * NT, 2), jnp.float32),
        ],
    )(view0, view1, proj_W, proj_b.reshape(NV, 1, DM), router_W, expert_keys)


def _dispatch_body(idx_ref, gv_ref, dest_ref, gk_ref, be_ref):
    ek = jnp.concatenate([idx_ref[:NT, :], idx_ref[NT:, :]], axis=1)
    gk = jnp.concatenate([gv_ref[:NT, :], gv_ref[NT:, :]], axis=1)
    iota_e = lax.broadcasted_iota(jnp.int32, (NT, NE), 1)

    cnt_row = jnp.zeros((NT, NE), jnp.float32)
    for c in range(NA):
        cnt_row = cnt_row + (ek[:, c:c + 1] == iota_e).astype(jnp.float32)

    ltri = (lax.broadcasted_iota(jnp.int32, (NT, NT), 0)
            > lax.broadcasted_iota(jnp.int32, (NT, NT), 1)).astype(jnp.float32)
    cbefore = jnp.dot(ltri, cnt_row, preferred_element_type=jnp.float32)

    cnt = jnp.sum(cnt_row, axis=0, keepdims=True)
    cp = jnp.floor((cnt + (BM - 1)) * (1.0 / BM)) * BM
    x = cp
    for s in (1, 2, 4):
        x = x + jnp.concatenate(
            [jnp.zeros((1, s), jnp.float32), x[:, :NE - s]], axis=1)
    pad_off = x - cp

    dest_cols = []
    for c in range(NA):
        sel = (ek[:, c:c + 1] == iota_e).astype(jnp.float32)
        base = jnp.sum(sel * (cbefore + pad_off), axis=1, keepdims=True)
        wr = jnp.zeros((NT, 1), jnp.float32)
        for cprev in range(c):
            wr = wr + (ek[:, cprev:cprev + 1] == ek[:, c:c + 1]).astype(jnp.float32)
        dest_cols.append(base + wr)
    dest_ref[...] = jnp.concatenate(dest_cols, axis=1).astype(jnp.int32)
    gk_ref[...] = gk

    ends = jnp.broadcast_to(pad_off + cp, (MAXB, NE))
    bio = lax.broadcasted_iota(jnp.int32, (MAXB, NE), 0).astype(jnp.float32) * BM
    be = jnp.sum((ends <= bio).astype(jnp.int32), axis=1, keepdims=True)
    nb_used = jnp.sum(cp * (1.0 / BM), axis=1, keepdims=True).astype(jnp.int32)
    be_ref[...] = jnp.concatenate([jnp.minimum(be, NE - 1), nb_used], axis=0)


def _dispatch(IDX, GV):
    return pl.pallas_call(
        _dispatch_body,
        out_shape=[
            jax.ShapeDtypeStruct((NT, NA), jnp.int32),
            jax.ShapeDtypeStruct((NT, NA), jnp.float32),
            jax.ShapeDtypeStruct((MAXB + 1, 1), jnp.int32),
        ],
    )(IDX, GV)


def _scatter_kernel(dest_hbm, gk_hbm, zi_hbm, zf_hbm, srcrow_hbm, gates_hbm,
                    dest_v, gk_v, sr_v, gt_v):
    wid = lax.axis_index("s") * 2 + lax.axis_index("c")

    @pl.when(wid == 0)
    def _():
        pltpu.sync_copy(dest_hbm, dest_v)
        pltpu.sync_copy(gk_hbm, gk_v)
        pltpu.sync_copy(zi_hbm, sr_v)
        pltpu.sync_copy(zf_hbm, gt_v)

        @plsc.parallel_loop(0, NTOT // 16, unroll=4)
        def _body(i):
            j0 = i * 16
            d = dest_v[pl.ds(j0, 16)]
            gt = gk_v[pl.ds(j0, 16)]
            j = lax.iota(jnp.int32, 16) + j0
            src = (j >> 2) + jnp.where((j & 3) >= 2, NT, 0)
            plsc.store_scatter(sr_v, [d], src)
            plsc.store_scatter(gt_v, [d], gt)

        pltpu.sync_copy(sr_v, srcrow_hbm)
        pltpu.sync_copy(gt_v, gates_hbm)


def _scatter(dest_flat, gk_flat):
    mesh = plsc.VectorSubcoreMesh(core_axis_name="c", subcore_axis_name="s")
    return pl.kernel(
        _scatter_kernel,
        mesh=mesh,
        compiler_params=pltpu.CompilerParams(needs_layout_passes=False),
        out_type=[
            jax.ShapeDtypeStruct((MAXR,), jnp.int32),
            jax.ShapeDtypeStruct((MAXR,), jnp.float32),
        ],
        scratch_types=[
            pltpu.VMEM((NTOT,), jnp.int32),
            pltpu.VMEM((NTOT,), jnp.float32),
            pltpu.VMEM((MAXR,), jnp.int32),
            pltpu.VMEM((MAXR,), jnp.float32),
        ],
    )(dest_flat, gk_flat,
      jnp.arange(MAXR, dtype=jnp.int32) % (NV * NT),
      jnp.zeros((MAXR,), jnp.float32))


def _make_gather(n_rows, n_src_rows, dtype=jnp.float32):
    per_w = n_rows // 32
    ch = 64
    n_ch = per_w // ch
    mesh = plsc.VectorSubcoreMesh(core_axis_name="c", subcore_axis_name="s")

    def body(idx_hbm, src_hbm, out_hbm, idx_v, buf0, buf1, sg0, sg1, sw0, sw1):
        wid = lax.axis_index("s") * 2 + lax.axis_index("c")
        base = wid * per_w
        pltpu.sync_copy(idx_hbm.at[pl.ds(base, per_w)], idx_v)
        bufs = (buf0, buf1)
        sgs = (sg0, sg1)
        sws = (sw0, sw1)

        def start_gather(q):
            return pltpu.async_copy(
                src_hbm.at[idx_v.at[pl.ds(q * ch, ch)]], bufs[q % 2], sgs[q % 2])

        def start_write(q):
            return pltpu.async_copy(
                bufs[q % 2], out_hbm.at[pl.ds(base + q * ch, ch)], sws[q % 2])

        dg = {0: start_gather(0)}
        dw = {}
        for q in range(n_ch):
            if q + 1 < n_ch:
                if q >= 1:
                    dw[q - 1].wait()
                dg[q + 1] = start_gather(q + 1)
            dg[q].wait()
            dw[q] = start_write(q)
        dw[n_ch - 1].wait()
        if n_ch >= 2:
            dw[n_ch - 2].wait()

    return pl.kernel(
        body,
        mesh=mesh,
        out_type=jax.ShapeDtypeStruct((n_rows, DM), dtype),
        scratch_types=[
            pltpu.VMEM((per_w,), jnp.int32),
            pltpu.VMEM((ch, DM), dtype),
            pltpu.VMEM((ch, DM), dtype),
            pltpu.SemaphoreType.DMA,
            pltpu.SemaphoreType.DMA,
            pltpu.SemaphoreType.DMA,
            pltpu.SemaphoreType.DMA,
        ],
    )


def _ffn_body(be_ref, xs_ref, w1_ref, b1_ref, w2_ref, b2_ref, g_ref, out_ref,
              w1b_ref, w2b_ref):
    b = pl.program_id(0)
    live = b < be_ref[MAXB]
    new_expert = jnp.logical_or(b == 0,
                                be_ref[b] != be_ref[jnp.maximum(b - 1, 0)])

    @pl.when(jnp.logical_and(live, new_expert))
    def _():
        w1b_ref[...] = w1_ref[0].astype(jnp.bfloat16)
        w2b_ref[...] = w2_ref[0].astype(jnp.bfloat16)

    @pl.when(live)
    def _():
        x = xs_ref[...].astype(jnp.bfloat16)
        mid = jnp.dot(x, w1b_ref[...],
                      preferred_element_type=jnp.float32) + b1_ref[0]
        mid = 0.5 * mid * (1.0 + lax.erf(mid * 0.7071067811865476))
        y = jnp.dot(mid.astype(jnp.bfloat16), w2b_ref[...],
                    preferred_element_type=jnp.float32) + b2_ref[0]
        out_ref[...] = g_ref[...] * y

    @pl.when(jnp.logical_not(live))
    def _():
        out_ref[...] = jnp.zeros_like(out_ref)


def _ffn(be, Xs, W1, b1, W2, b2, gates):
    grid_spec = pltpu.PrefetchScalarGridSpec(
        num_scalar_prefetch=1,
        grid=(MAXB,),
        in_specs=[
            pl.BlockSpec((BM, DM), lambda b, be_ref: (b, 0)),
            pl.BlockSpec((1, DM, DF), lambda b, be_ref: (be_ref[b], 0, 0)),
            pl.BlockSpec((1, 1, DF), lambda b, be_ref: (be_ref[b], 0, 0)),
            pl.BlockSpec((1, DF, DM), lambda b, be_ref: (be_ref[b], 0, 0)),
            pl.BlockSpec((1, 1, DM), lambda b, be_ref: (be_ref[b], 0, 0)),
            pl.BlockSpec((BM, 1), lambda b, be_ref: (b, 0)),
        ],
        out_specs=pl.BlockSpec((BM, DM), lambda b, be_ref: (b, 0)),
        scratch_shapes=[
            pltpu.VMEM((DM, DF), jnp.bfloat16),
            pltpu.VMEM((DF, DM), jnp.bfloat16),
        ],
    )
    return pl.pallas_call(
        _ffn_body,
        grid_spec=grid_spec,
        out_shape=jax.ShapeDtypeStruct((MAXR, DM), jnp.float32),
    )(be, Xs, W1, b1.reshape(NE, 1, DF), W2, b2.reshape(NE, 1, DM),
      gates.reshape(MAXR, 1))


def _combine_sc(dest_flat, Ys):
    tpw = NT // 32
    tch = 16
    rch = tch * NA
    n_ch = tpw // tch
    mesh = plsc.VectorSubcoreMesh(core_axis_name="c", subcore_axis_name="s")

    def body(idx_hbm, ys_hbm, out_hbm, idx_v, buf0, buf1, ov0, ov1,
             sg0, sg1, sw0, sw1):
        wid = lax.axis_index("s") * 2 + lax.axis_index("c")
        t0 = wid * tpw
        pltpu.sync_copy(idx_hbm.at[pl.ds(t0 * NA, tpw * NA)], idx_v)
        bufs = (buf0, buf1)
        ovs = (ov0, ov1)
        sgs = (sg0, sg1)
        sws = (sw0, sw1)

        def start_gather(q):
            return pltpu.async_copy(
                ys_hbm.at[idx_v.at[pl.ds(q * rch, rch)]], bufs[q % 2], sgs[q % 2])

        dg = {0: start_gather(0)}
        dw = {}
        for q in range(n_ch):
            if q + 1 < n_ch:
                dg[q + 1] = start_gather(q + 1)
            dg[q].wait()
            if q >= 2:
                dw[q - 2].wait()
            buf = bufs[q % 2]
            ov = ovs[q % 2]
            for tt in range(tch):
                @plsc.parallel_loop(0, DM // 16, unroll=4)
                def _add(l):
                    sl = pl.ds(l * 16, 16)
                    ov[tt, sl] = (buf[NA * tt, sl] + buf[NA * tt + 1, sl]
                                  + buf[NA * tt + 2, sl] + buf[NA * tt + 3, sl])
            dw[q] = pltpu.async_copy(
                ov, out_hbm.at[pl.ds(t0 + q * tch, tch)], sws[q % 2])
        for q in (n_ch - 2, n_ch - 1):
            dw[q].wait()

    return pl.kernel(
        body,
        mesh=mesh,
        compiler_params=pltpu.CompilerParams(needs_layout_passes=False),
        out_type=jax.ShapeDtypeStruct((NT, DM), jnp.float32),
        scratch_types=[
            pltpu.VMEM((tpw * NA,), jnp.int32),
            pltpu.VMEM((rch, DM), jnp.float32),
            pltpu.VMEM((rch, DM), jnp.float32),
            pltpu.VMEM((tch, DM), jnp.float32),
            pltpu.VMEM((tch, DM), jnp.float32),
            pltpu.SemaphoreType.DMA,
            pltpu.SemaphoreType.DMA,
            pltpu.SemaphoreType.DMA,
            pltpu.SemaphoreType.DMA,
        ],
    )(dest_flat, Ys)


def kernel(view0, view1, proj_W, proj_b, router_W, expert_keys, W1, b1, W2, b2):
    H, IDX, GV = _routing(view0, view1, proj_W, proj_b, router_W, expert_keys)
    dest, gk, be = _dispatch(IDX, GV)
    dest_flat = dest.reshape(NTOT)
    srcrow, gates = _scatter(dest_flat, gk.reshape(NTOT))
    Xs = _make_gather(MAXR, NV * NT)(srcrow, H)
    Ys = _ffn(be.reshape(MAXB + 1), Xs, W1, b1, W2, b2, gates)
    out = _combine_sc(dest_flat, Ys)
    return out.reshape(1, NT, DM)


# speedup vs baseline: 1.7567x; 1.0037x over previous
import functools

import jax
import jax.numpy as jnp
from jax import lax
from jax.experimental import pallas as pl
from jax.experimental.pallas import tpu as pltpu
from jax.experimental.pallas import tpu_sc as plsc

DM = 768
DF = 3072
NE = 8
NT = 2048
NV = 2
NA = 4
NTOT = NT * NA
BM = 512
MAXB = NTOT // BM + NE
MAXR = MAXB * BM
NB = NT // BM
BR = 512
NBR = NT // BR


def _routing_body(v0_ref, v1_ref, pw_ref, pb_ref, rw_ref, keys_ref,
                  h_ref, i_ref, g_ref):
    vi = pl.program_id(1)
    v = jnp.where(vi == 0, v0_ref[0], v1_ref[0])
    h = jnp.dot(v, pw_ref[vi], preferred_element_type=jnp.float32) + pb_ref[vi]
    r = jnp.dot(h, rw_ref[vi], preferred_element_type=jnp.float32)
    keys = keys_ref[...]
    d2_cols = []
    for e in range(NE):
        diff = r - keys[e:e + 1, :]
        d2_cols.append(jnp.sum(diff * diff, axis=1, keepdims=True))
    d2 = jnp.concatenate(d2_cols, axis=1)
    logits = -jnp.sqrt(d2 + 1e-12)
    iota = lax.broadcasted_iota(jnp.int32, (BR, NE), 1)
    m1 = jnp.max(logits, axis=1, keepdims=True)
    i1 = jnp.min(jnp.where(logits == m1, iota, NE), axis=1, keepdims=True)
    l2 = jnp.where(iota == i1, -1e30, logits)
    m2 = jnp.max(l2, axis=1, keepdims=True)
    i2 = jnp.min(jnp.where(l2 == m2, iota, NE), axis=1, keepdims=True)
    e2 = jnp.exp(m2 - m1)
    den = 1.0 + e2
    h_ref[...] = h
    i_ref[...] = jnp.concatenate([i1, i2], axis=1)
    g_ref[...] = jnp.concatenate([1.0 / den, e2 / den], axis=1)


def _routing(view0, view1, proj_W, proj_b, router_W, expert_keys):
    return pl.pallas_call(
        _routing_body,
        grid=(NBR, NV),
        in_specs=[
            pl.BlockSpec((1, BR, DM), lambda t, v: (0, t, 0)),
            pl.BlockSpec((1, BR, DM), lambda t, v: (0, t, 0)),
            pl.BlockSpec((NV, DM, DM), lambda t, v: (0, 0, 0)),
            pl.BlockSpec((NV, 1, DM), lambda t, v: (0, 0, 0)),
            pl.BlockSpec((NV, DM, NE), lambda t, v: (0, 0, 0)),
            pl.BlockSpec((NE, NE), lambda t, v: (0, 0)),
        ],
        out_specs=[
            pl.BlockSpec((BR, DM), lambda t, v: (v * NBR + t, 0)),
            pl.BlockSpec((BR, 2), lambda t, v: (v * NBR + t, 0)),
            pl.BlockSpec((BR, 2), lambda t, v: (v * NBR + t, 0)),
        ],
        out_shape=[
            jax.ShapeDtypeStruct((NV * NT, DM), jnp.float32),
            jax.ShapeDtypeStruct((NV * NT, 2), jnp.int32),
            jax.ShapeDtypeStruct((NV * NT, 2), jnp.float32),
        ],
    )(view0, view1, proj_W, proj_b.reshape(NV, 1, DM), router_W, expert_keys)


def _dispatch_body(idx_ref, gv_ref, dest_ref, gk_ref, be_ref):
    ek = jnp.concatenate([idx_ref[:NT, :], idx_ref[NT:, :]], axis=1)
    gk = jnp.concatenate([gv_ref[:NT, :], gv_ref[NT:, :]], axis=1)
    iota_e = lax.broadcasted_iota(jnp.int32, (NT, NE), 1)

    cnt_row = jnp.zeros((NT, NE), jnp.float32)
    for c in range(NA):
        cnt_row = cnt_row + (ek[:, c:c + 1] == iota_e).astype(jnp.float32)

    ltri = (lax.broadcasted_iota(jnp.int32, (NT, NT), 0)
            > lax.broadcasted_iota(jnp.int32, (NT, NT), 1)).astype(jnp.float32)
    cbefore = jnp.dot(ltri, cnt_row, preferred_element_type=jnp.float32)

    cnt = jnp.sum(cnt_row, axis=0, keepdims=True)
    cp = jnp.floor((cnt + (BM - 1)) * (1.0 / BM)) * BM
    x = cp
    for s in (1, 2, 4):
        x = x + jnp.concatenate(
            [jnp.zeros((1, s), jnp.float32), x[:, :NE - s]], axis=1)
    pad_off = x - cp

    dest_cols = []
    for c in range(NA):
        sel = (ek[:, c:c + 1] == iota_e).astype(jnp.float32)
        base = jnp.sum(sel * (cbefore + pad_off), axis=1, keepdims=True)
        wr = jnp.zeros((NT, 1), jnp.float32)
        for cprev in range(c):
            wr = wr + (ek[:, cprev:cprev + 1] == ek[:, c:c + 1]).astype(jnp.float32)
        dest_cols.append(base + wr)
    dest_ref[...] = jnp.concatenate(dest_cols, axis=1).astype(jnp.int32)
    gk_ref[...] = gk

    ends = jnp.broadcast_to(pad_off + cp, (MAXB, NE))
    bio = lax.broadcasted_iota(jnp.int32, (MAXB, NE), 0).astype(jnp.float32) * BM
    be = jnp.sum((ends <= bio).astype(jnp.int32), axis=1, keepdims=True)
    nb_used = jnp.sum(cp * (1.0 / BM), axis=1, keepdims=True).astype(jnp.int32)
    be_ref[...] = jnp.concatenate([jnp.minimum(be, NE - 1), nb_used], axis=0)


def _dispatch(IDX, GV):
    return pl.pallas_call(
        _dispatch_body,
        out_shape=[
            jax.ShapeDtypeStruct((NT, NA), jnp.int32),
            jax.ShapeDtypeStruct((NT, NA), jnp.float32),
            jax.ShapeDtypeStruct((MAXB + 1, 1), jnp.int32),
        ],
    )(IDX, GV)


def _scatter_kernel(dest_hbm, gk_hbm, zi_hbm, zf_hbm, srcrow_hbm, gates_hbm,
                    dest_v, gk_v, sr_v, gt_v):
    wid = lax.axis_index("s") * 2 + lax.axis_index("c")

    @pl.when(wid == 0)
    def _():
        pltpu.sync_copy(dest_hbm, dest_v)
        pltpu.sync_copy(gk_hbm, gk_v)
        pltpu.sync_copy(zi_hbm, sr_v)
        pltpu.sync_copy(zf_hbm, gt_v)

        @plsc.parallel_loop(0, NTOT // 16, unroll=4)
        def _body(i):
            j0 = i * 16
            d = dest_v[pl.ds(j0, 16)]
            gt = gk_v[pl.ds(j0, 16)]
            j = lax.iota(jnp.int32, 16) + j0
            src = (j >> 2) + jnp.where((j & 3) >= 2, NT, 0)
            plsc.store_scatter(sr_v, [d], src)
            plsc.store_scatter(gt_v, [d], gt)

        pltpu.sync_copy(sr_v, srcrow_hbm)
        pltpu.sync_copy(gt_v, gates_hbm)


def _scatter(dest_flat, gk_flat):
    mesh = plsc.VectorSubcoreMesh(core_axis_name="c", subcore_axis_name="s")
    return pl.kernel(
        _scatter_kernel,
        mesh=mesh,
        compiler_params=pltpu.CompilerParams(needs_layout_passes=False),
        out_type=[
            jax.ShapeDtypeStruct((MAXR,), jnp.int32),
            jax.ShapeDtypeStruct((MAXR,), jnp.float32),
        ],
        scratch_types=[
            pltpu.VMEM((NTOT,), jnp.int32),
            pltpu.VMEM((NTOT,), jnp.float32),
            pltpu.VMEM((MAXR,), jnp.int32),
            pltpu.VMEM((MAXR,), jnp.float32),
        ],
    )(dest_flat, gk_flat,
      jnp.arange(MAXR, dtype=jnp.int32) % (NV * NT),
      jnp.zeros((MAXR,), jnp.float32))


def _make_gather(n_rows, n_src_rows, dtype=jnp.float32):
    per_w = n_rows // 32
    ch = 64
    n_ch = per_w // ch
    mesh = plsc.VectorSubcoreMesh(core_axis_name="c", subcore_axis_name="s")

    def body(idx_hbm, src_hbm, out_hbm, idx_v, buf0, buf1, sg0, sg1, sw0, sw1):
        wid = lax.axis_index("s") * 2 + lax.axis_index("c")
        base = wid * per_w
        pltpu.sync_copy(idx_hbm.at[pl.ds(base, per_w)], idx_v)
        bufs = (buf0, buf1)
        sgs = (sg0, sg1)
        sws = (sw0, sw1)

        def start_gather(q):
            return pltpu.async_copy(
                src_hbm.at[idx_v.at[pl.ds(q * ch, ch)]], bufs[q % 2], sgs[q % 2])

        def start_write(q):
            return pltpu.async_copy(
                bufs[q % 2], out_hbm.at[pl.ds(base + q * ch, ch)], sws[q % 2])

        dg = {0: start_gather(0)}
        dw = {}
        for q in range(n_ch):
            if q + 1 < n_ch:
                if q >= 1:
                    dw[q - 1].wait()
                dg[q + 1] = start_gather(q + 1)
            dg[q].wait()
            dw[q] = start_write(q)
        dw[n_ch - 1].wait()
        if n_ch >= 2:
            dw[n_ch - 2].wait()

    return pl.kernel(
        body,
        mesh=mesh,
        out_type=jax.ShapeDtypeStruct((n_rows, DM), dtype),
        scratch_types=[
            pltpu.VMEM((per_w,), jnp.int32),
            pltpu.VMEM((ch, DM), dtype),
            pltpu.VMEM((ch, DM), dtype),
            pltpu.SemaphoreType.DMA,
            pltpu.SemaphoreType.DMA,
            pltpu.SemaphoreType.DMA,
            pltpu.SemaphoreType.DMA,
        ],
    )


def _ffn_body(be_ref, xs_ref, w1_ref, b1_ref, w2_ref, b2_ref, g_ref, out_ref,
              w1b_ref, w2b_ref):
    b = pl.program_id(0)
    live = b < be_ref[MAXB]
    new_expert = jnp.logical_or(b == 0,
                                be_ref[b] != be_ref[jnp.maximum(b - 1, 0)])

    @pl.when(jnp.logical_and(live, new_expert))
    def _():
        w1b_ref[...] = w1_ref[0].astype(jnp.bfloat16)
        w2b_ref[...] = w2_ref[0].astype(jnp.bfloat16)

    @pl.when(live)
    def _():
        x = xs_ref[...].astype(jnp.bfloat16)
        mid = jnp.dot(x, w1b_ref[...],
                      preferred_element_type=jnp.float32) + b1_ref[0]
        mid = 0.5 * mid * (1.0 + lax.erf(mid * 0.7071067811865476))
        y = jnp.dot(mid.astype(jnp.bfloat16), w2b_ref[...],
                    preferred_element_type=jnp.float32) + b2_ref[0]
        out_ref[...] = g_ref[...] * y

    @pl.when(jnp.logical_not(live))
    def _():
        out_ref[...] = jnp.zeros_like(out_ref)


def _ffn(be, Xs, W1, b1, W2, b2, gates):
    grid_spec = pltpu.PrefetchScalarGridSpec(
        num_scalar_prefetch=1,
        grid=(MAXB,),
        in_specs=[
            pl.BlockSpec((BM, DM), lambda b, be_ref: (b, 0)),
            pl.BlockSpec((1, DM, DF), lambda b, be_ref: (be_ref[b], 0, 0)),
            pl.BlockSpec((1, 1, DF), lambda b, be_ref: (be_ref[b], 0, 0)),
            pl.BlockSpec((1, DF, DM), lambda b, be_ref: (be_ref[b], 0, 0)),
            pl.BlockSpec((1, 1, DM), lambda b, be_ref: (be_ref[b], 0, 0)),
            pl.BlockSpec((BM, 1), lambda b, be_ref: (b, 0)),
        ],
        out_specs=pl.BlockSpec((BM, DM), lambda b, be_ref: (b, 0)),
        scratch_shapes=[
            pltpu.VMEM((DM, DF), jnp.bfloat16),
            pltpu.VMEM((DF, DM), jnp.bfloat16),
        ],
    )
    return pl.pallas_call(
        _ffn_body,
        grid_spec=grid_spec,
        out_shape=jax.ShapeDtypeStruct((MAXR, DM), jnp.float32),
    )(be, Xs, W1, b1.reshape(NE, 1, DF), W2, b2.reshape(NE, 1, DM),
      gates.reshape(MAXR, 1))


def _combine_sc(dest_flat, Ys):
    tpw = NT // 32
    tch = 16
    rch = tch * NA
    n_ch = tpw // tch
    mesh = plsc.VectorSubcoreMesh(core_axis_name="c", subcore_axis_name="s")

    def body(idx_hbm, ys_hbm, out_hbm, idx_v, buf0, buf1, ov0, ov1,
             sg0, sg1, sw0, sw1):
        wid = lax.axis_index("s") * 2 + lax.axis_index("c")
        t0 = wid * tpw
        pltpu.sync_copy(idx_hbm.at[pl.ds(t0 * NA, tpw * NA)], idx_v)
        bufs = (buf0, buf1)
        ovs = (ov0, ov1)
        sgs = (sg0, sg1)
        sws = (sw0, sw1)

        def start_gather(q):
            return pltpu.async_copy(
                ys_hbm.at[idx_v.at[pl.ds(q * rch, rch)]], bufs[q % 2], sgs[q % 2])

        dg = {0: start_gather(0)}
        dw = {}
        for q in range(n_ch):
            if q + 1 < n_ch:
                dg[q + 1] = start_gather(q + 1)
            dg[q].wait()
            if q >= 2:
                dw[q - 2].wait()
            buf = bufs[q % 2]
            ov = ovs[q % 2]
            for tt in range(tch):
                @plsc.parallel_loop(0, DM // 16, unroll=4)
                def _add(l):
                    sl = pl.ds(l * 16, 16)
                    ov[tt, sl] = (buf[NA * tt, sl] + buf[NA * tt + 1, sl]
                                  + buf[NA * tt + 2, sl] + buf[NA * tt + 3, sl])
            dw[q] = pltpu.async_copy(
                ov, out_hbm.at[pl.ds(t0 + q * tch, tch)], sws[q % 2])
        for q in (n_ch - 2, n_ch - 1):
            dw[q].wait()

    return pl.kernel(
        body,
        mesh=mesh,
        compiler_params=pltpu.CompilerParams(needs_layout_passes=False),
        out_type=jax.ShapeDtypeStruct((NT, DM), jnp.float32),
        scratch_types=[
            pltpu.VMEM((tpw * NA,), jnp.int32),
            pltpu.VMEM((rch, DM), jnp.float32),
            pltpu.VMEM((rch, DM), jnp.float32),
            pltpu.VMEM((tch, DM), jnp.float32),
            pltpu.VMEM((tch, DM), jnp.float32),
            pltpu.SemaphoreType.DMA,
            pltpu.SemaphoreType.DMA,
            pltpu.SemaphoreType.DMA,
            pltpu.SemaphoreType.DMA,
        ],
    )(dest_flat, Ys)


def kernel(view0, view1, proj_W, proj_b, router_W, expert_keys, W1, b1, W2, b2):
    H, IDX, GV = _routing(view0, view1, proj_W, proj_b, router_W, expert_keys)
    dest, gk, be = _dispatch(IDX, GV)
    dest_flat = dest.reshape(NTOT)
    srcrow, gates = _scatter(dest_flat, gk.reshape(NTOT))
    Xs = _make_gather(MAXR, NV * NT)(srcrow, H)
    Ys = _ffn(be.reshape(MAXB + 1), Xs, W1, b1, W2, b2, gates)
    out = _combine_sc(dest_flat, Ys)
    return out.reshape(1, NT, DM)


# speedup vs baseline: 1.7583x; 1.0009x over previous
import functools

import jax
import jax.numpy as jnp
from jax import lax
from jax.experimental import pallas as pl
from jax.experimental.pallas import tpu as pltpu
from jax.experimental.pallas import tpu_sc as plsc

DM = 768
DF = 3072
NE = 8
NT = 2048
NV = 2
NA = 4
NTOT = NT * NA
BM = 512
MAXB = NTOT // BM + NE
MAXR = MAXB * BM
NB = NT // BM
BR = 512
NBR = NT // BR


def _routing_body(v0_ref, v1_ref, pw_ref, pb_ref, rw_ref, keys_ref,
                  h_ref, i_ref, g_ref):
    vi = pl.program_id(1)
    v = jnp.where(vi == 0, v0_ref[0], v1_ref[0])
    h = jnp.dot(v, pw_ref[vi], preferred_element_type=jnp.float32) + pb_ref[vi]
    r = jnp.dot(h, rw_ref[vi], preferred_element_type=jnp.float32)
    keys = keys_ref[...]
    d2_cols = []
    for e in range(NE):
        diff = r - keys[e:e + 1, :]
        d2_cols.append(jnp.sum(diff * diff, axis=1, keepdims=True))
    d2 = jnp.concatenate(d2_cols, axis=1)
    logits = -jnp.sqrt(d2 + 1e-12)
    iota = lax.broadcasted_iota(jnp.int32, (BR, NE), 1)
    m1 = jnp.max(logits, axis=1, keepdims=True)
    i1 = jnp.min(jnp.where(logits == m1, iota, NE), axis=1, keepdims=True)
    l2 = jnp.where(iota == i1, -1e30, logits)
    m2 = jnp.max(l2, axis=1, keepdims=True)
    i2 = jnp.min(jnp.where(l2 == m2, iota, NE), axis=1, keepdims=True)
    e2 = jnp.exp(m2 - m1)
    den = 1.0 + e2
    h_ref[...] = h
    i_ref[...] = jnp.concatenate([i1, i2], axis=1)
    g_ref[...] = jnp.concatenate([1.0 / den, e2 / den], axis=1)


def _routing(view0, view1, proj_W, proj_b, router_W, expert_keys):
    return pl.pallas_call(
        _routing_body,
        grid=(NBR, NV),
        in_specs=[
            pl.BlockSpec((1, BR, DM), lambda t, v: (0, t, 0)),
            pl.BlockSpec((1, BR, DM), lambda t, v: (0, t, 0)),
            pl.BlockSpec((NV, DM, DM), lambda t, v: (0, 0, 0)),
            pl.BlockSpec((NV, 1, DM), lambda t, v: (0, 0, 0)),
            pl.BlockSpec((NV, DM, NE), lambda t, v: (0, 0, 0)),
            pl.BlockSpec((NE, NE), lambda t, v: (0, 0)),
        ],
        out_specs=[
            pl.BlockSpec((BR, DM), lambda t, v: (v * NBR + t, 0)),
            pl.BlockSpec((BR, 2), lambda t, v: (v * NBR + t, 0)),
            pl.BlockSpec((BR, 2), lambda t, v: (v * NBR + t, 0)),
        ],
        out_shape=[
            jax.ShapeDtypeStruct((NV * NT, DM), jnp.float32),
            jax.ShapeDtypeStruct((NV * NT, 2), jnp.int32),
            jax.ShapeDtypeStruct((NV * NT, 2), jnp.float32),
        ],
    )(view0, view1, proj_W, proj_b.reshape(NV, 1, DM), router_W, expert_keys)


def _dispatch_body(idx_ref, gv_ref, dest_ref, gk_ref, be_ref):
    ek = jnp.concatenate([idx_ref[:NT, :], idx_ref[NT:, :]], axis=1)
    gk = jnp.concatenate([gv_ref[:NT, :], gv_ref[NT:, :]], axis=1)
    iota_e = lax.broadcasted_iota(jnp.int32, (NT, NE), 1)

    cnt_row = jnp.zeros((NT, NE), jnp.float32)
    for c in range(NA):
        cnt_row = cnt_row + (ek[:, c:c + 1] == iota_e).astype(jnp.float32)

    ltri = (lax.broadcasted_iota(jnp.int32, (NT, NT), 0)
            > lax.broadcasted_iota(jnp.int32, (NT, NT), 1)).astype(jnp.bfloat16)
    cbefore = jnp.dot(ltri, cnt_row.astype(jnp.bfloat16),
                      preferred_element_type=jnp.float32)

    cnt = jnp.sum(cnt_row, axis=0, keepdims=True)
    cp = jnp.floor((cnt + (BM - 1)) * (1.0 / BM)) * BM
    x = cp
    for s in (1, 2, 4):
        x = x + jnp.concatenate(
            [jnp.zeros((1, s), jnp.float32), x[:, :NE - s]], axis=1)
    pad_off = x - cp

    dest_cols = []
    for c in range(NA):
        sel = (ek[:, c:c + 1] == iota_e).astype(jnp.float32)
        base = jnp.sum(sel * (cbefore + pad_off), axis=1, keepdims=True)
        wr = jnp.zeros((NT, 1), jnp.float32)
        for cprev in range(c):
            wr = wr + (ek[:, cprev:cprev + 1] == ek[:, c:c + 1]).astype(jnp.float32)
        dest_cols.append(base + wr)
    dest_ref[...] = jnp.concatenate(dest_cols, axis=1).astype(jnp.int32)
    gk_ref[...] = gk

    ends = jnp.broadcast_to(pad_off + cp, (MAXB, NE))
    bio = lax.broadcasted_iota(jnp.int32, (MAXB, NE), 0).astype(jnp.float32) * BM
    be = jnp.sum((ends <= bio).astype(jnp.int32), axis=1, keepdims=True)
    nb_used = jnp.sum(cp * (1.0 / BM), axis=1, keepdims=True).astype(jnp.int32)
    be_ref[...] = jnp.concatenate([jnp.minimum(be, NE - 1), nb_used], axis=0)


def _dispatch(IDX, GV):
    return pl.pallas_call(
        _dispatch_body,
        out_shape=[
            jax.ShapeDtypeStruct((NT, NA), jnp.int32),
            jax.ShapeDtypeStruct((NT, NA), jnp.float32),
            jax.ShapeDtypeStruct((MAXB + 1, 1), jnp.int32),
        ],
    )(IDX, GV)


def _scatter_kernel(dest_hbm, gk_hbm, zi_hbm, zf_hbm, srcrow_hbm, gates_hbm,
                    dest_v, gk_v, sr_v, gt_v):
    wid = lax.axis_index("s") * 2 + lax.axis_index("c")

    @pl.when(wid == 0)
    def _():
        pltpu.sync_copy(dest_hbm, dest_v)
        pltpu.sync_copy(gk_hbm, gk_v)
        pltpu.sync_copy(zi_hbm, sr_v)
        pltpu.sync_copy(zf_hbm, gt_v)

        @plsc.parallel_loop(0, NTOT // 16, unroll=4)
        def _body(i):
            j0 = i * 16
            d = dest_v[pl.ds(j0, 16)]
            gt = gk_v[pl.ds(j0, 16)]
            j = lax.iota(jnp.int32, 16) + j0
            src = (j >> 2) + jnp.where((j & 3) >= 2, NT, 0)
            plsc.store_scatter(sr_v, [d], src)
            plsc.store_scatter(gt_v, [d], gt)

        pltpu.sync_copy(sr_v, srcrow_hbm)
        pltpu.sync_copy(gt_v, gates_hbm)


def _scatter(dest_flat, gk_flat):
    mesh = plsc.VectorSubcoreMesh(core_axis_name="c", subcore_axis_name="s")
    return pl.kernel(
        _scatter_kernel,
        mesh=mesh,
        compiler_params=pltpu.CompilerParams(needs_layout_passes=False),
        out_type=[
            jax.ShapeDtypeStruct((MAXR,), jnp.int32),
            jax.ShapeDtypeStruct((MAXR,), jnp.float32),
        ],
        scratch_types=[
            pltpu.VMEM((NTOT,), jnp.int32),
            pltpu.VMEM((NTOT,), jnp.float32),
            pltpu.VMEM((MAXR,), jnp.int32),
            pltpu.VMEM((MAXR,), jnp.float32),
        ],
    )(dest_flat, gk_flat,
      jnp.arange(MAXR, dtype=jnp.int32) % (NV * NT),
      jnp.zeros((MAXR,), jnp.float32))


def _make_gather(n_rows, n_src_rows, dtype=jnp.float32):
    per_w = n_rows // 32
    ch = 64
    n_ch = per_w // ch
    mesh = plsc.VectorSubcoreMesh(core_axis_name="c", subcore_axis_name="s")

    def body(idx_hbm, src_hbm, out_hbm, idx_v, buf0, buf1, sg0, sg1, sw0, sw1):
        wid = lax.axis_index("s") * 2 + lax.axis_index("c")
        base = wid * per_w
        pltpu.sync_copy(idx_hbm.at[pl.ds(base, per_w)], idx_v)
        bufs = (buf0, buf1)
        sgs = (sg0, sg1)
        sws = (sw0, sw1)

        def start_gather(q):
            return pltpu.async_copy(
                src_hbm.at[idx_v.at[pl.ds(q * ch, ch)]], bufs[q % 2], sgs[q % 2])

        def start_write(q):
            return pltpu.async_copy(
                bufs[q % 2], out_hbm.at[pl.ds(base + q * ch, ch)], sws[q % 2])

        dg = {0: start_gather(0)}
        dw = {}
        for q in range(n_ch):
            if q + 1 < n_ch:
                if q >= 1:
                    dw[q - 1].wait()
                dg[q + 1] = start_gather(q + 1)
            dg[q].wait()
            dw[q] = start_write(q)
        dw[n_ch - 1].wait()
        if n_ch >= 2:
            dw[n_ch - 2].wait()

    return pl.kernel(
        body,
        mesh=mesh,
        out_type=jax.ShapeDtypeStruct((n_rows, DM), dtype),
        scratch_types=[
            pltpu.VMEM((per_w,), jnp.int32),
            pltpu.VMEM((ch, DM), dtype),
            pltpu.VMEM((ch, DM), dtype),
            pltpu.SemaphoreType.DMA,
            pltpu.SemaphoreType.DMA,
            pltpu.SemaphoreType.DMA,
            pltpu.SemaphoreType.DMA,
        ],
    )


def _ffn_body(be_ref, xs_ref, w1a_ref, w1b_ref, b1_ref, w2a_ref, w2b_ref,
              b2_ref, g_ref, out_ref):
    b = pl.program_id(0)
    live = b < be_ref[MAXB]

    @pl.when(live)
    def _():
        x = xs_ref[...].astype(jnp.bfloat16)
        mid = jnp.concatenate(
            [jnp.dot(x, w1a_ref[0].astype(jnp.bfloat16),
                     preferred_element_type=jnp.float32),
             jnp.dot(x, w1b_ref[0].astype(jnp.bfloat16),
                     preferred_element_type=jnp.float32)], axis=1) + b1_ref[0]
        mid = 0.5 * mid * (1.0 + lax.erf(mid * 0.7071067811865476))
        mid = mid.astype(jnp.bfloat16)
        y = (jnp.dot(mid[:, :DF // 2], w2a_ref[0].astype(jnp.bfloat16),
                     preferred_element_type=jnp.float32)
             + jnp.dot(mid[:, DF // 2:], w2b_ref[0].astype(jnp.bfloat16),
                       preferred_element_type=jnp.float32)) + b2_ref[0]
        out_ref[...] = g_ref[...] * y

    @pl.when(jnp.logical_not(live))
    def _():
        out_ref[...] = jnp.zeros_like(out_ref)


def _ffn(be, Xs, W1, b1, W2, b2, gates):
    grid_spec = pltpu.PrefetchScalarGridSpec(
        num_scalar_prefetch=1,
        grid=(MAXB,),
        in_specs=[
            pl.BlockSpec((BM, DM), lambda b, be_ref: (b, 0)),
            pl.BlockSpec((1, DM, DF // 2), lambda b, be_ref: (be_ref[b], 0, 0)),
            pl.BlockSpec((1, DM, DF // 2), lambda b, be_ref: (be_ref[b], 0, 1)),
            pl.BlockSpec((1, 1, DF), lambda b, be_ref: (be_ref[b], 0, 0)),
            pl.BlockSpec((1, DF // 2, DM), lambda b, be_ref: (be_ref[b], 0, 0)),
            pl.BlockSpec((1, DF // 2, DM), lambda b, be_ref: (be_ref[b], 1, 0)),
            pl.BlockSpec((1, 1, DM), lambda b, be_ref: (be_ref[b], 0, 0)),
            pl.BlockSpec((BM, 1), lambda b, be_ref: (b, 0)),
        ],
        out_specs=pl.BlockSpec((BM, DM), lambda b, be_ref: (b, 0)),
    )
    return pl.pallas_call(
        _ffn_body,
        grid_spec=grid_spec,
        out_shape=jax.ShapeDtypeStruct((MAXR, DM), jnp.float32),
    )(be, Xs, W1, W1, b1.reshape(NE, 1, DF), W2, W2, b2.reshape(NE, 1, DM),
      gates.reshape(MAXR, 1))


def _combine_sc(dest_flat, Ys):
    tpw = NT // 32
    tch = 16
    rch = tch * NA
    n_ch = tpw // tch
    mesh = plsc.VectorSubcoreMesh(core_axis_name="c", subcore_axis_name="s")

    def body(idx_hbm, ys_hbm, out_hbm, idx_v, buf0, buf1, ov0, ov1,
             sg0, sg1, sw0, sw1):
        wid = lax.axis_index("s") * 2 + lax.axis_index("c")
        t0 = wid * tpw
        pltpu.sync_copy(idx_hbm.at[pl.ds(t0 * NA, tpw * NA)], idx_v)
        bufs = (buf0, buf1)
        ovs = (ov0, ov1)
        sgs = (sg0, sg1)
        sws = (sw0, sw1)

        def start_gather(q):
            return pltpu.async_copy(
                ys_hbm.at[idx_v.at[pl.ds(q * rch, rch)]], bufs[q % 2], sgs[q % 2])

        dg = {0: start_gather(0)}
        dw = {}
        for q in range(n_ch):
            if q + 1 < n_ch:
                dg[q + 1] = start_gather(q + 1)
            dg[q].wait()
            if q >= 2:
                dw[q - 2].wait()
            buf = bufs[q % 2]
            ov = ovs[q % 2]
            for tt in range(tch):
                @plsc.parallel_loop(0, DM // 16, unroll=4)
                def _add(l):
                    sl = pl.ds(l * 16, 16)
                    ov[tt, sl] = (buf[NA * tt, sl] + buf[NA * tt + 1, sl]
                                  + buf[NA * tt + 2, sl] + buf[NA * tt + 3, sl])
            dw[q] = pltpu.async_copy(
                ov, out_hbm.at[pl.ds(t0 + q * tch, tch)], sws[q % 2])
        for q in (n_ch - 2, n_ch - 1):
            dw[q].wait()

    return pl.kernel(
        body,
        mesh=mesh,
        compiler_params=pltpu.CompilerParams(needs_layout_passes=False),
        out_type=jax.ShapeDtypeStruct((NT, DM), jnp.float32),
        scratch_types=[
            pltpu.VMEM((tpw * NA,), jnp.int32),
            pltpu.VMEM((rch, DM), jnp.float32),
            pltpu.VMEM((rch, DM), jnp.float32),
            pltpu.VMEM((tch, DM), jnp.float32),
            pltpu.VMEM((tch, DM), jnp.float32),
            pltpu.SemaphoreType.DMA,
            pltpu.SemaphoreType.DMA,
            pltpu.SemaphoreType.DMA,
            pltpu.SemaphoreType.DMA,
        ],
    )(dest_flat, Ys)


def kernel(view0, view1, proj_W, proj_b, router_W, expert_keys, W1, b1, W2, b2):
    H, IDX, GV = _routing(view0, view1, proj_W, proj_b, router_W, expert_keys)
    dest, gk, be = _dispatch(IDX, GV)
    dest_flat = dest.reshape(NTOT)
    srcrow, gates = _scatter(dest_flat, gk.reshape(NTOT))
    Xs = _make_gather(MAXR, NV * NT)(srcrow, H)
    Ys = _ffn(be.reshape(MAXB + 1), Xs, W1, b1, W2, b2, gates)
    out = _combine_sc(dest_flat, Ys)
    return out.reshape(1, NT, DM)
